# baseline (speedup 1.0000x reference)
import functools

import jax
import jax.numpy as jnp
from jax import lax
from jax.experimental import pallas as pl
from jax.experimental.pallas import tpu as pltpu
from jax.experimental.pallas import tpu_sc as plsc

N = 10000
E = 320000
EMB = 128
NP = 10240
NC = 2
NS = 16
NW = NC * NS
CH = 80
G = 125
NCH = NW * G
RPT = NP // NS
AUG = 32
BN_EPS_EDGE = 1e-6
BN_EPS_OUT = 1e-5


def _deg_body(idx_ref, out_ref, ones_v, stage_v, idx_v, deg_sh):
    c = lax.axis_index("c")
    s = lax.axis_index("s")
    wid = s * NC + c

    for i in range(CH // 16):
        ones_v[pl.ds(i * 16, 16)] = jnp.ones((16,), jnp.float32)

    def zero_stage(i, carry):
        stage_v[pl.ds(i * 16, 16)] = jnp.zeros((16,), jnp.float32)
        return carry
    lax.fori_loop(0, RPT // 16, zero_stage, 0)
    pltpu.sync_copy(stage_v, deg_sh.at[pl.ds(s * RPT, RPT)])
    plsc.subcore_barrier()

    def chunk(g, carry):
        cid = wid * G + g
        pltpu.sync_copy(idx_ref.at[cid], idx_v)
        pltpu.sync_copy(ones_v, deg_sh.at[idx_v.at[0]], add=True)
        return carry
    lax.fori_loop(0, G, chunk, 0)
    plsc.subcore_barrier()

    pltpu.sync_copy(deg_sh.at[pl.ds(s * RPT, RPT)], stage_v)
    pltpu.sync_copy(stage_v, out_ref.at[pl.ds(c * NP + s * RPT, RPT)])


def _deg_call(idx_all):
    mesh = plsc.VectorSubcoreMesh(core_axis_name="c", subcore_axis_name="s")
    f = pl.kernel(
        _deg_body,
        out_type=jax.ShapeDtypeStruct((NC * NP,), jnp.float32),
        mesh=mesh,
        compiler_params=pltpu.CompilerParams(needs_layout_passes=False),
        scratch_types=[
            pltpu.VMEM((CH,), jnp.float32),
            pltpu.VMEM((RPT,), jnp.float32),
            pltpu.VMEM((2, CH), jnp.int32),
            pltpu.VMEM_SHARED((NP,), jnp.float32),
        ],
    )
    return f(idx_all)


def _prep_body(idx_ref, ea_ref, dinv_ref, out_ref,
               dinv_v, idxA, idxB, eaA, eaB, augA, augB,
               semiA, semiB, semoA, semoB):
    c = lax.axis_index("c")
    s = lax.axis_index("s")
    wid = s * NC + c

    pltpu.sync_copy(dinv_ref, dinv_v)
    lane0 = jnp.where(lax.iota(jnp.int32, 16) == 0, 1.0, 0.0)

    def issue(g, idxv, eav, semi):
        cid = wid * G + g
        pltpu.sync_copy(idx_ref.at[cid], idxv)
        pltpu.async_copy(ea_ref.at[pl.ds(cid * CH, CH)], eav, semi)

    def process(g, idxv, eav, augv, semi, semo):
        cid = wid * G + g
        pltpu.make_async_copy(ea_ref.at[pl.ds(0, CH)], eav, semi).wait()

        @pl.when(g >= 2)
        def _():
            pltpu.make_async_copy(
                augv, out_ref.at[pl.ds(cid * CH, CH)], semo).wait()
        for k in range(CH // 16):
            ir = idxv[0, pl.ds(k * 16, 16)]
            dr = plsc.load_gather(dinv_v, [ir])
            for t in range(16):
                sc = dr[t]
                e = k * 16 + t
                augv[e, pl.ds(0, 16)] = eav[e, pl.ds(0, 16)] * sc
                augv[e, pl.ds(16, 16)] = lane0 * sc
        pltpu.async_copy(augv, out_ref.at[pl.ds(cid * CH, CH)], semo)

    issue(0, idxA, eaA, semiA)
    issue(1, idxB, eaB, semiB)

    def body(gg, carry):
        g0 = 2 * gg
        process(g0, idxA, eaA, augA, semiA, semoA)

        @pl.when(g0 + 2 < G)
        def _():
            issue(g0 + 2, idxA, eaA, semiA)

        process(g0 + 1, idxB, eaB, augB, semiB, semoB)

        @pl.when(g0 + 3 < G)
        def _():
            issue(g0 + 3, idxB, eaB, semiB)
        return carry
    lax.fori_loop(0, (G - 1) // 2, body, 0)
    process(G - 1, idxA, eaA, augA, semiA, semoA)
    pltpu.make_async_copy(augA, out_ref.at[pl.ds(0, CH)], semoA).wait()
    pltpu.make_async_copy(augB, out_ref.at[pl.ds(0, CH)], semoB).wait()


def _prep_call(idx_all, edge_attr, dinv):
    mesh = plsc.VectorSubcoreMesh(core_axis_name="c", subcore_axis_name="s")
    f = pl.kernel(
        _prep_body,
        out_type=jax.ShapeDtypeStruct((E, AUG), jnp.float32),
        mesh=mesh,
        compiler_params=pltpu.CompilerParams(needs_layout_passes=False),
        scratch_types=[
            pltpu.VMEM((NP,), jnp.float32),
            pltpu.VMEM((2, CH), jnp.int32),
            pltpu.VMEM((2, CH), jnp.int32),
            pltpu.VMEM((CH, 16), jnp.float32),
            pltpu.VMEM((CH, 16), jnp.float32),
            pltpu.VMEM((CH, AUG), jnp.float32),
            pltpu.VMEM((CH, AUG), jnp.float32),
            pltpu.SemaphoreType.DMA,
            pltpu.SemaphoreType.DMA,
            pltpu.SemaphoreType.DMA,
            pltpu.SemaphoreType.DMA,
        ],
    )
    return f(idx_all, edge_attr, dinv)


def _edge_body(idx_ref, x_ref, ee_ref, out_ref,
               idxA, idxB, xrA, xrB, eeA, eeB,
               semxA, semxB, semeA, semeB, aggr_sh):
    c = lax.axis_index("c")
    s = lax.axis_index("s")
    wid = s * NC + c

    def zrow(i, carry):
        for j in range(EMB // 16):
            xrA[i, pl.ds(j * 16, 16)] = jnp.zeros((16,), jnp.float32)
        return carry
    lax.fori_loop(0, CH, zrow, 0)

    def zslab(k, carry):
        pltpu.sync_copy(xrA, aggr_sh.at[pl.ds(s * RPT + k * CH, CH)])
        return carry
    lax.fori_loop(0, RPT // CH, zslab, 0)
    plsc.subcore_barrier()

    def issue(g, idxv, xrv, eev, semx, seme):
        cid = wid * G + g
        pltpu.sync_copy(idx_ref.at[cid], idxv)
        pltpu.async_copy(x_ref.at[idxv.at[0]], xrv, semx)
        pltpu.async_copy(ee_ref.at[pl.ds(cid * (CH // 2), CH // 2)], eev, seme)

    def process(idxv, xrv, eev, semx, seme):
        pltpu.make_async_copy(x_ref.at[idxv.at[0]], xrv, semx).wait()
        pltpu.make_async_copy(ee_ref.at[pl.ds(0, CH // 2)], eev, seme).wait()

        def pairrow(r, icarry):
            for j in range(EMB // 16):
                pw = eev[r, pl.ds(j * 16, 16)]
                pb = plsc.bitcast(pw, jnp.bfloat16)
                ea_, eb_ = plsc.unpack(pb, format=plsc.PackFormat.INTERLEAVED)
                xa = xrv[2 * r, pl.ds(j * 16, 16)]
                xb = xrv[2 * r + 1, pl.ds(j * 16, 16)]
                xrv[2 * r, pl.ds(j * 16, 16)] = jnp.maximum(xa + ea_, 0.0)
                xrv[2 * r + 1, pl.ds(j * 16, 16)] = jnp.maximum(xb + eb_, 0.0)
            return icarry
        lax.fori_loop(0, CH // 2, pairrow, 0)
        pltpu.sync_copy(xrv, aggr_sh.at[idxv.at[1]], add=True)

    issue(0, idxA, xrA, eeA, semxA, semeA)
    issue(1, idxB, xrB, eeB, semxB, semeB)

    def body(gg, carry):
        g0 = 2 * gg
        process(idxA, xrA, eeA, semxA, semeA)

        @pl.when(g0 + 2 < G)
        def _():
            issue(g0 + 2, idxA, xrA, eeA, semxA, semeA)

        process(idxB, xrB, eeB, semxB, semeB)

        @pl.when(g0 + 3 < G)
        def _():
            issue(g0 + 3, idxB, xrB, eeB, semxB, semeB)
        return carry
    lax.fori_loop(0, (G - 1) // 2, body, 0)
    process(idxA, xrA, eeA, semxA, semeA)

    plsc.subcore_barrier()

    def wout(k, carry):
        off = s * RPT + k * CH
        pltpu.sync_copy(aggr_sh.at[pl.ds(off, CH)], xrA)
        pltpu.sync_copy(xrA, out_ref.at[c, pl.ds(off, CH)])
        return carry
    lax.fori_loop(0, RPT // CH, wout, 0)


def _edge_call(idx_all, xs, ee):
    mesh = plsc.VectorSubcoreMesh(core_axis_name="c", subcore_axis_name="s")
    f = pl.kernel(
        _edge_body,
        out_type=jax.ShapeDtypeStruct((NC, NP, EMB), jnp.float32),
        mesh=mesh,
        compiler_params=pltpu.CompilerParams(needs_layout_passes=False),
        scratch_types=[
            pltpu.VMEM((2, CH), jnp.int32),
            pltpu.VMEM((2, CH), jnp.int32),
            pltpu.VMEM((CH, EMB), jnp.float32),
            pltpu.VMEM((CH, EMB), jnp.float32),
            pltpu.VMEM((CH // 2, EMB), jnp.float32),
            pltpu.VMEM((CH // 2, EMB), jnp.float32),
            pltpu.SemaphoreType.DMA,
            pltpu.SemaphoreType.DMA,
            pltpu.SemaphoreType.DMA,
            pltpu.SemaphoreType.DMA,
            pltpu.VMEM_SHARED((NP, EMB), jnp.float32),
        ],
    )
    return f(idx_all, xs, ee)


_BLK = 1024
_EBLK = 1280


def _enc_kernel(ax_ref, af_ref, degp_ref, wae_ref, bae_ref, wp1_ref, wp2_ref,
                bp_ref, h_ref, dinv_ref):
    x2 = jnp.dot(af_ref[...], wae_ref[...],
                 preferred_element_type=jnp.float32) + bae_ref[...]
    h_ref[...] = (jnp.dot(ax_ref[...], wp1_ref[...],
                          preferred_element_type=jnp.float32)
                  + jnp.dot(x2, wp2_ref[...],
                            preferred_element_type=jnp.float32)
                  + bp_ref[...])
    deg = degp_ref[0:1, :] + degp_ref[1:2, :] + 1.0
    dinv_ref[...] = lax.rsqrt(deg)


def _enc_call(ax_p, af_p, degp, W_ae, b_ae, Wp1, Wp2, b_proj):
    full = lambda shape: pl.BlockSpec(shape, lambda i: (0,) * len(shape))
    return pl.pallas_call(
        _enc_kernel,
        grid=(NP // _BLK,),
        in_specs=[
            pl.BlockSpec((_BLK, EMB), lambda i: (i, 0)),
            pl.BlockSpec((_BLK, 16), lambda i: (i, 0)),
            pl.BlockSpec((2, _BLK), lambda i: (0, i)),
            full((16, EMB)), full((1, EMB)), full((EMB, EMB)),
            full((EMB, EMB)), full((1, EMB)),
        ],
        out_specs=[
            pl.BlockSpec((_BLK, EMB), lambda i: (i, 0)),
            pl.BlockSpec((1, _BLK), lambda i: (0, i)),
        ],
        out_shape=[
            jax.ShapeDtypeStruct((NP, EMB), jnp.float32),
            jax.ShapeDtypeStruct((1, NP), jnp.float32),
        ],
    )(ax_p, af_p, degp, W_ae, b_ae, Wp1, Wp2, b_proj)


def _xs_kernel(h_ref, dinvb_ref, wl_ref, bl_ref, x_ref, xs_ref):
    x = jnp.dot(h_ref[...], wl_ref[...],
                preferred_element_type=jnp.float32) + bl_ref[...]
    x_ref[...] = x
    xs_ref[...] = x * dinvb_ref[...]


def _xs_call(h, dinvb, wl, bl):
    full = lambda shape: pl.BlockSpec(shape, lambda i: (0,) * len(shape))
    return pl.pallas_call(
        _xs_kernel,
        grid=(NP // _BLK,),
        in_specs=[
            pl.BlockSpec((_BLK, EMB), lambda i: (i, 0)),
            pl.BlockSpec((_BLK, EMB), lambda i: (i, 0)),
            full((EMB, EMB)), full((1, EMB)),
        ],
        out_specs=[pl.BlockSpec((_BLK, EMB), lambda i: (i, 0)),
                   pl.BlockSpec((_BLK, EMB), lambda i: (i, 0))],
        out_shape=[jax.ShapeDtypeStruct((NP, EMB), jnp.float32),
                   jax.ShapeDtypeStruct((NP, EMB), jnp.float32)],
    )(h, dinvb, wl, bl)


def _ee_kernel(eaA_ref, eaB_ref, wbe_ref, bbe_ref, g_ref, b_ref, m_ref,
               v_ref, ee_ref):
    scale = g_ref[...] * lax.rsqrt(v_ref[...] + BN_EPS_EDGE)
    w = wbe_ref[...] * scale
    cst = (bbe_ref[...] - m_ref[...]) * scale + b_ref[...]
    waug = jnp.concatenate([w, cst, jnp.zeros((AUG - 17, EMB), jnp.float32)],
                           axis=0)
    eeA = jnp.dot(eaA_ref[...], waug, preferred_element_type=jnp.float32)
    eeB = jnp.dot(eaB_ref[...], waug, preferred_element_type=jnp.float32)
    a16 = lax.bitcast_convert_type(eeA.astype(jnp.bfloat16), jnp.int16)
    b16 = lax.bitcast_convert_type(eeB.astype(jnp.bfloat16), jnp.int16)
    word = ((b16.astype(jnp.int32) << 16)
            | (a16.astype(jnp.int32) & 0xFFFF))
    ee_ref[...] = lax.bitcast_convert_type(word, jnp.float32)


def _ee_call(ea_aug, wbe, bbe, g, b, m, v):
    full = lambda shape: pl.BlockSpec(shape, lambda i: (0,) * len(shape))
    nblk = E // 2 // _EBLK
    return pl.pallas_call(
        _ee_kernel,
        grid=(nblk,),
        in_specs=[
            pl.BlockSpec((_EBLK, AUG), lambda i: (i, 0)),
            pl.BlockSpec((_EBLK, AUG), lambda i: (i + nblk, 0)),
            full((16, EMB)), full((1, EMB)), full((1, EMB)),
            full((1, EMB)), full((1, EMB)), full((1, EMB)),
        ],
        out_specs=pl.BlockSpec((_EBLK, EMB), lambda i: (i, 0)),
        out_shape=jax.ShapeDtypeStruct((E // 2, EMB), jnp.float32),
    )(ea_aug, ea_aug, wbe, bbe, g, b, m, v)


def _upd_kernel(relu_out,
                aggp_ref, x_ref, dinvb_ref, d2_ref, wih_ref, whh_ref, bih_ref,
                bhh_ref, root_ref, bng_ref, bnb_ref, bnm_ref, bnv_ref, h_ref):
    x = x_ref[...]
    aggr = (aggp_ref[0] + aggp_ref[1]) * dinvb_ref[...]
    gi = jnp.dot(aggr, wih_ref[...],
                 preferred_element_type=jnp.float32) + bih_ref[...]
    gh = jnp.dot(x, whh_ref[...],
                 preferred_element_type=jnp.float32) + bhh_ref[...]
    r = jax.nn.sigmoid(gi[:, :EMB] + gh[:, :EMB])
    z = jax.nn.sigmoid(gi[:, EMB:2 * EMB] + gh[:, EMB:2 * EMB])
    nn_ = jnp.tanh(gi[:, 2 * EMB:] + r * gh[:, 2 * EMB:])
    upd = (1.0 - z) * nn_ + z * x
    conv = upd + jnp.maximum(x + root_ref[...], 0.0) * d2_ref[...]
    hb = ((conv - bnm_ref[...]) * lax.rsqrt(bnv_ref[...] + BN_EPS_OUT)
          * bng_ref[...] + bnb_ref[...])
    if relu_out:
        hb = jnp.maximum(hb, 0.0)
    h_ref[...] = hb


def _upd_call(aggp, x, dinvb, d2b, wih, whh, bih, bhh, root, bng, bnb,
              bnm, bnv, relu_out):
    full = lambda shape: pl.BlockSpec(shape, lambda i: (0,) * len(shape))
    return pl.pallas_call(
        functools.partial(_upd_kernel, relu_out),
        grid=(NP // _BLK,),
        in_specs=[
            pl.BlockSpec((2, _BLK, EMB), lambda i: (0, i, 0)),
            pl.BlockSpec((_BLK, EMB), lambda i: (i, 0)),
            pl.BlockSpec((_BLK, EMB), lambda i: (i, 0)),
            pl.BlockSpec((_BLK, EMB), lambda i: (i, 0)),
            full((EMB, 3 * EMB)), full((EMB, 3 * EMB)),
            full((1, 3 * EMB)), full((1, 3 * EMB)),
            full((1, EMB)), full((1, EMB)), full((1, EMB)),
            full((1, EMB)), full((1, EMB)),
        ],
        out_specs=pl.BlockSpec((_BLK, EMB), lambda i: (i, 0)),
        out_shape=jax.ShapeDtypeStruct((NP, EMB), jnp.float32),
    )(aggp, x, dinvb, d2b, wih, whh, bih, bhh, root, bng, bnb, bnm, bnv)


def kernel(atom_x, atom_feature, edge_index, edge_attr,
           W_ae, b_ae, W_proj, b_proj,
           W_lin, b_lin, root_emb, W_be, b_be,
           bn_be_g, bn_be_b, bn_be_m, bn_be_v,
           W_ih, W_hh, b_ih, b_hh,
           bn_g, bn_b, bn_m, bn_v):
    pad_n = NP - N

    idx_all = jnp.concatenate(
        [edge_index[0].reshape(NCH, 1, CH), edge_index[1].reshape(NCH, 1, CH)],
        axis=1)

    def _pair(v):
        a = v[:E // 2].reshape(NCH, CH // 2, 1)
        b = v[E // 2:].reshape(NCH, CH // 2, 1)
        return jnp.concatenate([a, b], axis=2).reshape(NCH, 1, CH)
    idx_pair = jnp.concatenate(
        [_pair(edge_index[0]), _pair(edge_index[1])], axis=1)
    ax_p = jnp.pad(atom_x, ((0, pad_n), (0, 0)))
    af_p = jnp.pad(atom_feature, ((0, pad_n), (0, 0)))

    r2 = lambda a: a.reshape(1, -1)
    Wp1 = W_proj[:EMB]
    Wp2 = W_proj[EMB:]

    degp = _deg_call(idx_all).reshape(NC, NP)
    h, dinv2d = _enc_call(ax_p, af_p, degp, W_ae, r2(b_ae), Wp1, Wp2,
                          r2(b_proj))
    dinv = dinv2d.reshape(NP)
    dinvb = jnp.broadcast_to(dinv[:, None], (NP, EMB))
    d2b = jnp.broadcast_to((dinv * dinv)[:, None], (NP, EMB))

    ea_aug = _prep_call(idx_all, edge_attr, dinv)
    ee0 = _ee_call(ea_aug, W_be[0], r2(b_be[0]), r2(bn_be_g[0]),
                   r2(bn_be_b[0]), r2(bn_be_m[0]), r2(bn_be_v[0]))
    ee1 = _ee_call(ea_aug, W_be[1], r2(b_be[1]), r2(bn_be_g[1]),
                   r2(bn_be_b[1]), r2(bn_be_m[1]), r2(bn_be_v[1]))

    x0, xs0 = _xs_call(h, dinvb, W_lin[0], r2(b_lin[0]))
    aggp0 = _edge_call(idx_pair, xs0, ee0)
    h1 = _upd_call(aggp0, x0, dinvb, d2b, W_ih[0], W_hh[0], r2(b_ih[0]),
                   r2(b_hh[0]), r2(root_emb[0]), r2(bn_g[0]), r2(bn_b[0]),
                   r2(bn_m[0]), r2(bn_v[0]), relu_out=True)

    x1, xs1 = _xs_call(h1, dinvb, W_lin[1], r2(b_lin[1]))
    aggp1 = _edge_call(idx_pair, xs1, ee1)
    h2 = _upd_call(aggp1, x1, dinvb, d2b, W_ih[1], W_hh[1], r2(b_ih[1]),
                   r2(b_hh[1]), r2(root_emb[1]), r2(bn_g[1]), r2(bn_b[1]),
                   r2(bn_m[1]), r2(bn_v[1]), relu_out=False)

    return h2[:N]


# speedup vs baseline: 1.3705x; 1.3705x over previous
import functools

import jax
import jax.numpy as jnp
from jax import lax
from jax.experimental import pallas as pl
from jax.experimental.pallas import tpu as pltpu
from jax.experimental.pallas import tpu_sc as plsc

N = 10000
E = 320000
EMB = 128
NP = 10240
NC = 2
NS = 16
NW = NC * NS
CH = 80
G = 125
NCH = NW * G
RPT = NP // NS
AUG = 32
BN_EPS_EDGE = 1e-6
BN_EPS_OUT = 1e-5


def _deg_body(idx_ref, out_ref, ones_v, stage_v, idx_v, deg_sh):
    c = lax.axis_index("c")
    s = lax.axis_index("s")
    wid = s * NC + c

    for i in range(CH // 16):
        ones_v[pl.ds(i * 16, 16)] = jnp.ones((16,), jnp.float32)

    def zero_stage(i, carry):
        stage_v[pl.ds(i * 16, 16)] = jnp.zeros((16,), jnp.float32)
        return carry
    lax.fori_loop(0, RPT // 16, zero_stage, 0)
    pltpu.sync_copy(stage_v, deg_sh.at[pl.ds(s * RPT, RPT)])
    plsc.subcore_barrier()

    def chunk(g, carry):
        cid = wid * G + g
        pltpu.sync_copy(idx_ref.at[cid], idx_v)
        pltpu.sync_copy(ones_v, deg_sh.at[idx_v.at[0]], add=True)
        return carry
    lax.fori_loop(0, G, chunk, 0)
    plsc.subcore_barrier()

    pltpu.sync_copy(deg_sh.at[pl.ds(s * RPT, RPT)], stage_v)
    pltpu.sync_copy(stage_v, out_ref.at[pl.ds(c * NP + s * RPT, RPT)])


def _deg_call(idx_all):
    mesh = plsc.VectorSubcoreMesh(core_axis_name="c", subcore_axis_name="s")
    f = pl.kernel(
        _deg_body,
        out_type=jax.ShapeDtypeStruct((NC * NP,), jnp.float32),
        mesh=mesh,
        compiler_params=pltpu.CompilerParams(needs_layout_passes=False),
        scratch_types=[
            pltpu.VMEM((CH,), jnp.float32),
            pltpu.VMEM((RPT,), jnp.float32),
            pltpu.VMEM((2, CH), jnp.int32),
            pltpu.VMEM_SHARED((NP,), jnp.float32),
        ],
    )
    return f(idx_all)


def _prep_body(idx_ref, ea_ref, dinv_ref, out_ref,
               dinv_v, idxA, idxB, eaA, eaB, augA, augB,
               semiA, semiB, semoA, semoB):
    c = lax.axis_index("c")
    s = lax.axis_index("s")
    wid = s * NC + c

    pltpu.sync_copy(dinv_ref, dinv_v)
    lane0 = jnp.where(lax.iota(jnp.int32, 16) == 0, 1.0, 0.0)

    def issue(g, idxv, eav, semi):
        cid = wid * G + g
        pltpu.sync_copy(idx_ref.at[cid], idxv)
        pltpu.async_copy(ea_ref.at[pl.ds(cid * CH, CH)], eav, semi)

    def process(g, idxv, eav, augv, semi, semo):
        cid = wid * G + g
        pltpu.make_async_copy(ea_ref.at[pl.ds(0, CH)], eav, semi).wait()

        @pl.when(g >= 2)
        def _():
            pltpu.make_async_copy(
                augv, out_ref.at[pl.ds(cid * CH, CH)], semo).wait()
        for k in range(CH // 16):
            ir = idxv[0, pl.ds(k * 16, 16)]
            dr = plsc.load_gather(dinv_v, [ir])
            for t in range(16):
                sc = dr[t]
                e = k * 16 + t
                augv[e, pl.ds(0, 16)] = eav[e, pl.ds(0, 16)] * sc
                augv[e, pl.ds(16, 16)] = lane0 * sc
        pltpu.async_copy(augv, out_ref.at[pl.ds(cid * CH, CH)], semo)

    issue(0, idxA, eaA, semiA)
    issue(1, idxB, eaB, semiB)

    def body(gg, carry):
        g0 = 2 * gg
        process(g0, idxA, eaA, augA, semiA, semoA)

        @pl.when(g0 + 2 < G)
        def _():
            issue(g0 + 2, idxA, eaA, semiA)

        process(g0 + 1, idxB, eaB, augB, semiB, semoB)

        @pl.when(g0 + 3 < G)
        def _():
            issue(g0 + 3, idxB, eaB, semiB)
        return carry
    lax.fori_loop(0, (G - 1) // 2, body, 0)
    process(G - 1, idxA, eaA, augA, semiA, semoA)
    pltpu.make_async_copy(augA, out_ref.at[pl.ds(0, CH)], semoA).wait()
    pltpu.make_async_copy(augB, out_ref.at[pl.ds(0, CH)], semoB).wait()


def _prep_call(idx_all, edge_attr, dinv):
    mesh = plsc.VectorSubcoreMesh(core_axis_name="c", subcore_axis_name="s")
    f = pl.kernel(
        _prep_body,
        out_type=jax.ShapeDtypeStruct((E, AUG), jnp.float32),
        mesh=mesh,
        compiler_params=pltpu.CompilerParams(needs_layout_passes=False),
        scratch_types=[
            pltpu.VMEM((NP,), jnp.float32),
            pltpu.VMEM((2, CH), jnp.int32),
            pltpu.VMEM((2, CH), jnp.int32),
            pltpu.VMEM((CH, 16), jnp.float32),
            pltpu.VMEM((CH, 16), jnp.float32),
            pltpu.VMEM((CH, AUG), jnp.float32),
            pltpu.VMEM((CH, AUG), jnp.float32),
            pltpu.SemaphoreType.DMA,
            pltpu.SemaphoreType.DMA,
            pltpu.SemaphoreType.DMA,
            pltpu.SemaphoreType.DMA,
        ],
    )
    return f(idx_all, edge_attr, dinv)


def _edge_body(idx_ref, x_ref, ee_ref, out_ref,
               idxA, idxB, xrA, xrB, eeA, eeB, msg_v,
               semxA, semxB, semeA, semeB, aggr_sh):
    c = lax.axis_index("c")
    s = lax.axis_index("s")
    wid = s * NC + c

    def zrow(i, carry):
        for j in range(EMB // 16):
            msg_v[i, pl.ds(j * 16, 16)] = jnp.zeros((16,), jnp.float32)
        return carry
    lax.fori_loop(0, CH, zrow, 0)

    def zslab(k, carry):
        pltpu.sync_copy(msg_v, aggr_sh.at[pl.ds(s * RPT + k * CH, CH)])
        return carry
    lax.fori_loop(0, RPT // CH, zslab, 0)
    plsc.subcore_barrier()

    def issue(g, idxv, xrv, eev, semx, seme):
        cid = wid * G + g
        pltpu.sync_copy(idx_ref.at[cid], idxv)
        pltpu.async_copy(x_ref.at[idxv.at[0]], xrv, semx)
        pltpu.async_copy(ee_ref.at[pl.ds(cid * (CH // 2), CH // 2)], eev, seme)

    def process(idxv, xrv, eev, semx, seme):
        pltpu.make_async_copy(x_ref.at[idxv.at[0]], xrv, semx).wait()
        pltpu.make_async_copy(ee_ref.at[pl.ds(0, CH // 2)], eev, seme).wait()

        @plsc.parallel_loop(0, CH // 2, unroll=2)
        def pairrow(r):
            for j in range(EMB // 16):
                pw = eev[r, pl.ds(j * 16, 16)]
                pb = plsc.bitcast(pw, jnp.bfloat16)
                ea_, eb_ = plsc.unpack(pb, format=plsc.PackFormat.INTERLEAVED)
                xa = xrv[2 * r, pl.ds(j * 16, 16)]
                xb = xrv[2 * r + 1, pl.ds(j * 16, 16)]
                msg_v[2 * r, pl.ds(j * 16, 16)] = jnp.maximum(xa + ea_, 0.0)
                msg_v[2 * r + 1, pl.ds(j * 16, 16)] = jnp.maximum(xb + eb_, 0.0)
        pltpu.sync_copy(msg_v, aggr_sh.at[idxv.at[1]], add=True)

    issue(0, idxA, xrA, eeA, semxA, semeA)
    issue(1, idxB, xrB, eeB, semxB, semeB)

    def body(gg, carry):
        g0 = 2 * gg
        process(idxA, xrA, eeA, semxA, semeA)

        @pl.when(g0 + 2 < G)
        def _():
            issue(g0 + 2, idxA, xrA, eeA, semxA, semeA)

        process(idxB, xrB, eeB, semxB, semeB)

        @pl.when(g0 + 3 < G)
        def _():
            issue(g0 + 3, idxB, xrB, eeB, semxB, semeB)
        return carry
    lax.fori_loop(0, (G - 1) // 2, body, 0)
    process(idxA, xrA, eeA, semxA, semeA)

    plsc.subcore_barrier()

    def wout(k, carry):
        off = s * RPT + k * CH
        pltpu.sync_copy(aggr_sh.at[pl.ds(off, CH)], msg_v)
        pltpu.sync_copy(msg_v, out_ref.at[c, pl.ds(off, CH)])
        return carry
    lax.fori_loop(0, RPT // CH, wout, 0)


def _edge_call(idx_all, xs, ee):
    mesh = plsc.VectorSubcoreMesh(core_axis_name="c", subcore_axis_name="s")
    f = pl.kernel(
        _edge_body,
        out_type=jax.ShapeDtypeStruct((NC, NP, EMB), jnp.float32),
        mesh=mesh,
        compiler_params=pltpu.CompilerParams(needs_layout_passes=False),
        scratch_types=[
            pltpu.VMEM((2, CH), jnp.int32),
            pltpu.VMEM((2, CH), jnp.int32),
            pltpu.VMEM((CH, EMB), jnp.float32),
            pltpu.VMEM((CH, EMB), jnp.float32),
            pltpu.VMEM((CH // 2, EMB), jnp.float32),
            pltpu.VMEM((CH // 2, EMB), jnp.float32),
            pltpu.VMEM((CH, EMB), jnp.float32),
            pltpu.SemaphoreType.DMA,
            pltpu.SemaphoreType.DMA,
            pltpu.SemaphoreType.DMA,
            pltpu.SemaphoreType.DMA,
            pltpu.VMEM_SHARED((NP, EMB), jnp.float32),
        ],
    )
    return f(idx_all, xs, ee)


_BLK = 1024
_EBLK = 1280


def _enc_kernel(ax_ref, af_ref, degp_ref, wae_ref, bae_ref, wp1_ref, wp2_ref,
                bp_ref, h_ref, dinv_ref):
    x2 = jnp.dot(af_ref[...], wae_ref[...],
                 preferred_element_type=jnp.float32) + bae_ref[...]
    h_ref[...] = (jnp.dot(ax_ref[...], wp1_ref[...],
                          preferred_element_type=jnp.float32)
                  + jnp.dot(x2, wp2_ref[...],
                            preferred_element_type=jnp.float32)
                  + bp_ref[...])
    deg = degp_ref[0:1, :] + degp_ref[1:2, :] + 1.0
    dinv_ref[...] = lax.rsqrt(deg)


def _enc_call(ax_p, af_p, degp, W_ae, b_ae, Wp1, Wp2, b_proj):
    full = lambda shape: pl.BlockSpec(shape, lambda i: (0,) * len(shape))
    return pl.pallas_call(
        _enc_kernel,
        grid=(NP // _BLK,),
        in_specs=[
            pl.BlockSpec((_BLK, EMB), lambda i: (i, 0)),
            pl.BlockSpec((_BLK, 16), lambda i: (i, 0)),
            pl.BlockSpec((2, _BLK), lambda i: (0, i)),
            full((16, EMB)), full((1, EMB)), full((EMB, EMB)),
            full((EMB, EMB)), full((1, EMB)),
        ],
        out_specs=[
            pl.BlockSpec((_BLK, EMB), lambda i: (i, 0)),
            pl.BlockSpec((1, _BLK), lambda i: (0, i)),
        ],
        out_shape=[
            jax.ShapeDtypeStruct((NP, EMB), jnp.float32),
            jax.ShapeDtypeStruct((1, NP), jnp.float32),
        ],
    )(ax_p, af_p, degp, W_ae, b_ae, Wp1, Wp2, b_proj)


def _xs_kernel(h_ref, dinvb_ref, wl_ref, bl_ref, x_ref, xs_ref):
    x = jnp.dot(h_ref[...], wl_ref[...],
                preferred_element_type=jnp.float32) + bl_ref[...]
    x_ref[...] = x
    xs_ref[...] = x * dinvb_ref[...]


def _xs_call(h, dinvb, wl, bl):
    full = lambda shape: pl.BlockSpec(shape, lambda i: (0,) * len(shape))
    return pl.pallas_call(
        _xs_kernel,
        grid=(NP // _BLK,),
        in_specs=[
            pl.BlockSpec((_BLK, EMB), lambda i: (i, 0)),
            pl.BlockSpec((_BLK, EMB), lambda i: (i, 0)),
            full((EMB, EMB)), full((1, EMB)),
        ],
        out_specs=[pl.BlockSpec((_BLK, EMB), lambda i: (i, 0)),
                   pl.BlockSpec((_BLK, EMB), lambda i: (i, 0))],
        out_shape=[jax.ShapeDtypeStruct((NP, EMB), jnp.float32),
                   jax.ShapeDtypeStruct((NP, EMB), jnp.float32)],
    )(h, dinvb, wl, bl)


def _ee_kernel(eaA_ref, eaB_ref, wbe_ref, bbe_ref, g_ref, b_ref, m_ref,
               v_ref, ee_ref):
    scale = g_ref[...] * lax.rsqrt(v_ref[...] + BN_EPS_EDGE)
    w = wbe_ref[...] * scale
    cst = (bbe_ref[...] - m_ref[...]) * scale + b_ref[...]
    waug = jnp.concatenate([w, cst, jnp.zeros((AUG - 17, EMB), jnp.float32)],
                           axis=0)
    eeA = jnp.dot(eaA_ref[...], waug, preferred_element_type=jnp.float32)
    eeB = jnp.dot(eaB_ref[...], waug, preferred_element_type=jnp.float32)
    a16 = lax.bitcast_convert_type(eeA.astype(jnp.bfloat16), jnp.int16)
    b16 = lax.bitcast_convert_type(eeB.astype(jnp.bfloat16), jnp.int16)
    word = ((b16.astype(jnp.int32) << 16)
            | (a16.astype(jnp.int32) & 0xFFFF))
    ee_ref[...] = lax.bitcast_convert_type(word, jnp.float32)


def _ee_call(ea_aug, wbe, bbe, g, b, m, v):
    full = lambda shape: pl.BlockSpec(shape, lambda i: (0,) * len(shape))
    nblk = E // 2 // _EBLK
    return pl.pallas_call(
        _ee_kernel,
        grid=(nblk,),
        in_specs=[
            pl.BlockSpec((_EBLK, AUG), lambda i: (i, 0)),
            pl.BlockSpec((_EBLK, AUG), lambda i: (i + nblk, 0)),
            full((16, EMB)), full((1, EMB)), full((1, EMB)),
            full((1, EMB)), full((1, EMB)), full((1, EMB)),
        ],
        out_specs=pl.BlockSpec((_EBLK, EMB), lambda i: (i, 0)),
        out_shape=jax.ShapeDtypeStruct((E // 2, EMB), jnp.float32),
    )(ea_aug, ea_aug, wbe, bbe, g, b, m, v)


def _upd_kernel(relu_out,
                aggp_ref, x_ref, dinvb_ref, d2_ref, wih_ref, whh_ref, bih_ref,
                bhh_ref, root_ref, bng_ref, bnb_ref, bnm_ref, bnv_ref, h_ref):
    x = x_ref[...]
    aggr = (aggp_ref[0] + aggp_ref[1]) * dinvb_ref[...]
    gi = jnp.dot(aggr, wih_ref[...],
                 preferred_element_type=jnp.float32) + bih_ref[...]
    gh = jnp.dot(x, whh_ref[...],
                 preferred_element_type=jnp.float32) + bhh_ref[...]
    r = jax.nn.sigmoid(gi[:, :EMB] + gh[:, :EMB])
    z = jax.nn.sigmoid(gi[:, EMB:2 * EMB] + gh[:, EMB:2 * EMB])
    nn_ = jnp.tanh(gi[:, 2 * EMB:] + r * gh[:, 2 * EMB:])
    upd = (1.0 - z) * nn_ + z * x
    conv = upd + jnp.maximum(x + root_ref[...], 0.0) * d2_ref[...]
    hb = ((conv - bnm_ref[...]) * lax.rsqrt(bnv_ref[...] + BN_EPS_OUT)
          * bng_ref[...] + bnb_ref[...])
    if relu_out:
        hb = jnp.maximum(hb, 0.0)
    h_ref[...] = hb


def _upd_call(aggp, x, dinvb, d2b, wih, whh, bih, bhh, root, bng, bnb,
              bnm, bnv, relu_out):
    full = lambda shape: pl.BlockSpec(shape, lambda i: (0,) * len(shape))
    return pl.pallas_call(
        functools.partial(_upd_kernel, relu_out),
        grid=(NP // _BLK,),
        in_specs=[
            pl.BlockSpec((2, _BLK, EMB), lambda i: (0, i, 0)),
            pl.BlockSpec((_BLK, EMB), lambda i: (i, 0)),
            pl.BlockSpec((_BLK, EMB), lambda i: (i, 0)),
            pl.BlockSpec((_BLK, EMB), lambda i: (i, 0)),
            full((EMB, 3 * EMB)), full((EMB, 3 * EMB)),
            full((1, 3 * EMB)), full((1, 3 * EMB)),
            full((1, EMB)), full((1, EMB)), full((1, EMB)),
            full((1, EMB)), full((1, EMB)),
        ],
        out_specs=pl.BlockSpec((_BLK, EMB), lambda i: (i, 0)),
        out_shape=jax.ShapeDtypeStruct((NP, EMB), jnp.float32),
    )(aggp, x, dinvb, d2b, wih, whh, bih, bhh, root, bng, bnb, bnm, bnv)


def kernel(atom_x, atom_feature, edge_index, edge_attr,
           W_ae, b_ae, W_proj, b_proj,
           W_lin, b_lin, root_emb, W_be, b_be,
           bn_be_g, bn_be_b, bn_be_m, bn_be_v,
           W_ih, W_hh, b_ih, b_hh,
           bn_g, bn_b, bn_m, bn_v):
    pad_n = NP - N

    idx_all = jnp.concatenate(
        [edge_index[0].reshape(NCH, 1, CH), edge_index[1].reshape(NCH, 1, CH)],
        axis=1)

    def _pair(v):
        a = v[:E // 2].reshape(NCH, CH // 2, 1)
        b = v[E // 2:].reshape(NCH, CH // 2, 1)
        return jnp.concatenate([a, b], axis=2).reshape(NCH, 1, CH)
    idx_pair = jnp.concatenate(
        [_pair(edge_index[0]), _pair(edge_index[1])], axis=1)
    ax_p = jnp.pad(atom_x, ((0, pad_n), (0, 0)))
    af_p = jnp.pad(atom_feature, ((0, pad_n), (0, 0)))

    r2 = lambda a: a.reshape(1, -1)
    Wp1 = W_proj[:EMB]
    Wp2 = W_proj[EMB:]

    degp = _deg_call(idx_all).reshape(NC, NP)
    h, dinv2d = _enc_call(ax_p, af_p, degp, W_ae, r2(b_ae), Wp1, Wp2,
                          r2(b_proj))
    dinv = dinv2d.reshape(NP)
    dinvb = jnp.broadcast_to(dinv[:, None], (NP, EMB))
    d2b = jnp.broadcast_to((dinv * dinv)[:, None], (NP, EMB))

    ea_aug = _prep_call(idx_all, edge_attr, dinv)
    ee0 = _ee_call(ea_aug, W_be[0], r2(b_be[0]), r2(bn_be_g[0]),
                   r2(bn_be_b[0]), r2(bn_be_m[0]), r2(bn_be_v[0]))
    ee1 = _ee_call(ea_aug, W_be[1], r2(b_be[1]), r2(bn_be_g[1]),
                   r2(bn_be_b[1]), r2(bn_be_m[1]), r2(bn_be_v[1]))

    x0, xs0 = _xs_call(h, dinvb, W_lin[0], r2(b_lin[0]))
    aggp0 = _edge_call(idx_pair, xs0, ee0)
    h1 = _upd_call(aggp0, x0, dinvb, d2b, W_ih[0], W_hh[0], r2(b_ih[0]),
                   r2(b_hh[0]), r2(root_emb[0]), r2(bn_g[0]), r2(bn_b[0]),
                   r2(bn_m[0]), r2(bn_v[0]), relu_out=True)

    x1, xs1 = _xs_call(h1, dinvb, W_lin[1], r2(b_lin[1]))
    aggp1 = _edge_call(idx_pair, xs1, ee1)
    h2 = _upd_call(aggp1, x1, dinvb, d2b, W_ih[1], W_hh[1], r2(b_ih[1]),
                   r2(b_hh[1]), r2(root_emb[1]), r2(bn_g[1]), r2(bn_b[1]),
                   r2(bn_m[1]), r2(bn_v[1]), relu_out=False)

    return h2[:N]


# speedup vs baseline: 1.3777x; 1.0052x over previous
import functools

import jax
import jax.numpy as jnp
from jax import lax
from jax.experimental import pallas as pl
from jax.experimental.pallas import tpu as pltpu
from jax.experimental.pallas import tpu_sc as plsc

N = 10000
E = 320000
EMB = 128
NP = 10240
NC = 2
NS = 16
NW = NC * NS
CH = 80
G = 125
NCH = NW * G
RPT = NP // NS
AUG = 32
BN_EPS_EDGE = 1e-6
BN_EPS_OUT = 1e-5


def _deg_body(idx_ref, out_ref, ones_v, stage_v, idx_v, deg_sh):
    c = lax.axis_index("c")
    s = lax.axis_index("s")
    wid = s * NC + c

    for i in range(CH // 16):
        ones_v[pl.ds(i * 16, 16)] = jnp.ones((16,), jnp.float32)

    def zero_stage(i, carry):
        stage_v[pl.ds(i * 16, 16)] = jnp.zeros((16,), jnp.float32)
        return carry
    lax.fori_loop(0, RPT // 16, zero_stage, 0)
    pltpu.sync_copy(stage_v, deg_sh.at[pl.ds(s * RPT, RPT)])
    plsc.subcore_barrier()

    def chunk(g, carry):
        cid = wid * G + g
        pltpu.sync_copy(idx_ref.at[cid], idx_v)
        pltpu.sync_copy(ones_v, deg_sh.at[idx_v.at[0]], add=True)
        return carry
    lax.fori_loop(0, G, chunk, 0)
    plsc.subcore_barrier()

    pltpu.sync_copy(deg_sh.at[pl.ds(s * RPT, RPT)], stage_v)
    pltpu.sync_copy(stage_v, out_ref.at[pl.ds(c * NP + s * RPT, RPT)])


def _deg_call(idx_all):
    mesh = plsc.VectorSubcoreMesh(core_axis_name="c", subcore_axis_name="s")
    f = pl.kernel(
        _deg_body,
        out_type=jax.ShapeDtypeStruct((NC * NP,), jnp.float32),
        mesh=mesh,
        compiler_params=pltpu.CompilerParams(needs_layout_passes=False),
        scratch_types=[
            pltpu.VMEM((CH,), jnp.float32),
            pltpu.VMEM((RPT,), jnp.float32),
            pltpu.VMEM((2, CH), jnp.int32),
            pltpu.VMEM_SHARED((NP,), jnp.float32),
        ],
    )
    return f(idx_all)


def _prep_body(idx_ref, ea_ref, dinv_ref, out_ref,
               dinv_v, idxA, idxB, eaA, eaB, augA, augB,
               semiA, semiB, semoA, semoB):
    c = lax.axis_index("c")
    s = lax.axis_index("s")
    wid = s * NC + c

    pltpu.sync_copy(dinv_ref, dinv_v)
    lane0 = jnp.where(lax.iota(jnp.int32, 16) == 0, 1.0, 0.0)

    def issue(g, idxv, eav, semi):
        cid = wid * G + g
        pltpu.sync_copy(idx_ref.at[cid], idxv)
        pltpu.async_copy(ea_ref.at[pl.ds(cid * CH, CH)], eav, semi)

    def process(g, idxv, eav, augv, semi, semo):
        cid = wid * G + g
        pltpu.make_async_copy(ea_ref.at[pl.ds(0, CH)], eav, semi).wait()

        @pl.when(g >= 2)
        def _():
            pltpu.make_async_copy(
                augv, out_ref.at[pl.ds(cid * CH, CH)], semo).wait()
        @plsc.parallel_loop(0, CH // 16, unroll=2)
        def scalegrp(k):
            ir = idxv[0, pl.ds(k * 16, 16)]
            dr = plsc.load_gather(dinv_v, [ir])
            for t in range(16):
                sc = dr[t]
                e = k * 16 + t
                augv[e, pl.ds(0, 16)] = eav[e, pl.ds(0, 16)] * sc
                augv[e, pl.ds(16, 16)] = lane0 * sc
        pltpu.async_copy(augv, out_ref.at[pl.ds(cid * CH, CH)], semo)

    issue(0, idxA, eaA, semiA)
    issue(1, idxB, eaB, semiB)

    def body(gg, carry):
        g0 = 2 * gg
        process(g0, idxA, eaA, augA, semiA, semoA)

        @pl.when(g0 + 2 < G)
        def _():
            issue(g0 + 2, idxA, eaA, semiA)

        process(g0 + 1, idxB, eaB, augB, semiB, semoB)

        @pl.when(g0 + 3 < G)
        def _():
            issue(g0 + 3, idxB, eaB, semiB)
        return carry
    lax.fori_loop(0, (G - 1) // 2, body, 0)
    process(G - 1, idxA, eaA, augA, semiA, semoA)
    pltpu.make_async_copy(augA, out_ref.at[pl.ds(0, CH)], semoA).wait()
    pltpu.make_async_copy(augB, out_ref.at[pl.ds(0, CH)], semoB).wait()


def _prep_call(idx_all, edge_attr, dinv):
    mesh = plsc.VectorSubcoreMesh(core_axis_name="c", subcore_axis_name="s")
    f = pl.kernel(
        _prep_body,
        out_type=jax.ShapeDtypeStruct((E, AUG), jnp.float32),
        mesh=mesh,
        compiler_params=pltpu.CompilerParams(needs_layout_passes=False),
        scratch_types=[
            pltpu.VMEM((NP,), jnp.float32),
            pltpu.VMEM((2, CH), jnp.int32),
            pltpu.VMEM((2, CH), jnp.int32),
            pltpu.VMEM((CH, 16), jnp.float32),
            pltpu.VMEM((CH, 16), jnp.float32),
            pltpu.VMEM((CH, AUG), jnp.float32),
            pltpu.VMEM((CH, AUG), jnp.float32),
            pltpu.SemaphoreType.DMA,
            pltpu.SemaphoreType.DMA,
            pltpu.SemaphoreType.DMA,
            pltpu.SemaphoreType.DMA,
        ],
    )
    return f(idx_all, edge_attr, dinv)


def _edge_body(idx_ref, x_ref, ee_ref, out_ref,
               idxA, idxB, xrA, xrB, eeA, eeB, msg_v,
               semxA, semxB, semeA, semeB, aggr_sh):
    c = lax.axis_index("c")
    s = lax.axis_index("s")
    wid = s * NC + c

    def zrow(i, carry):
        for j in range(EMB // 16):
            msg_v[i, pl.ds(j * 16, 16)] = jnp.zeros((16,), jnp.float32)
        return carry
    lax.fori_loop(0, CH, zrow, 0)

    def zslab(k, carry):
        pltpu.sync_copy(msg_v, aggr_sh.at[pl.ds(s * RPT + k * CH, CH)])
        return carry
    lax.fori_loop(0, RPT // CH, zslab, 0)
    plsc.subcore_barrier()

    def issue(g, idxv, xrv, eev, semx, seme):
        cid = wid * G + g
        pltpu.sync_copy(idx_ref.at[cid], idxv)
        pltpu.async_copy(x_ref.at[idxv.at[0]], xrv, semx)
        pltpu.async_copy(ee_ref.at[pl.ds(cid * (CH // 2), CH // 2)], eev, seme)

    def process(idxv, xrv, eev, semx, seme):
        pltpu.make_async_copy(x_ref.at[idxv.at[0]], xrv, semx).wait()
        pltpu.make_async_copy(ee_ref.at[pl.ds(0, CH // 2)], eev, seme).wait()

        @plsc.parallel_loop(0, CH // 2, unroll=4)
        def pairrow(r):
            for j in range(EMB // 16):
                pw = eev[r, pl.ds(j * 16, 16)]
                pb = plsc.bitcast(pw, jnp.bfloat16)
                ea_, eb_ = plsc.unpack(pb, format=plsc.PackFormat.INTERLEAVED)
                xa = xrv[2 * r, pl.ds(j * 16, 16)]
                xb = xrv[2 * r + 1, pl.ds(j * 16, 16)]
                msg_v[2 * r, pl.ds(j * 16, 16)] = jnp.maximum(xa + ea_, 0.0)
                msg_v[2 * r + 1, pl.ds(j * 16, 16)] = jnp.maximum(xb + eb_, 0.0)
        pltpu.sync_copy(msg_v, aggr_sh.at[idxv.at[1]], add=True)

    issue(0, idxA, xrA, eeA, semxA, semeA)
    issue(1, idxB, xrB, eeB, semxB, semeB)

    def body(gg, carry):
        g0 = 2 * gg
        process(idxA, xrA, eeA, semxA, semeA)

        @pl.when(g0 + 2 < G)
        def _():
            issue(g0 + 2, idxA, xrA, eeA, semxA, semeA)

        process(idxB, xrB, eeB, semxB, semeB)

        @pl.when(g0 + 3 < G)
        def _():
            issue(g0 + 3, idxB, xrB, eeB, semxB, semeB)
        return carry
    lax.fori_loop(0, (G - 1) // 2, body, 0)
    process(idxA, xrA, eeA, semxA, semeA)

    plsc.subcore_barrier()

    def wout(k, carry):
        off = s * RPT + k * CH
        pltpu.sync_copy(aggr_sh.at[pl.ds(off, CH)], msg_v)
        pltpu.sync_copy(msg_v, out_ref.at[c, pl.ds(off, CH)])
        return carry
    lax.fori_loop(0, RPT // CH, wout, 0)


def _edge_call(idx_all, xs, ee):
    mesh = plsc.VectorSubcoreMesh(core_axis_name="c", subcore_axis_name="s")
    f = pl.kernel(
        _edge_body,
        out_type=jax.ShapeDtypeStruct((NC, NP, EMB), jnp.float32),
        mesh=mesh,
        compiler_params=pltpu.CompilerParams(needs_layout_passes=False),
        scratch_types=[
            pltpu.VMEM((2, CH), jnp.int32),
            pltpu.VMEM((2, CH), jnp.int32),
            pltpu.VMEM((CH, EMB), jnp.float32),
            pltpu.VMEM((CH, EMB), jnp.float32),
            pltpu.VMEM((CH // 2, EMB), jnp.float32),
            pltpu.VMEM((CH // 2, EMB), jnp.float32),
            pltpu.VMEM((CH, EMB), jnp.float32),
            pltpu.SemaphoreType.DMA,
            pltpu.SemaphoreType.DMA,
            pltpu.SemaphoreType.DMA,
            pltpu.SemaphoreType.DMA,
            pltpu.VMEM_SHARED((NP, EMB), jnp.float32),
        ],
    )
    return f(idx_all, xs, ee)


_BLK = 1024
_EBLK = 1280


def _enc_kernel(ax_ref, af_ref, degp_ref, wae_ref, bae_ref, wp1_ref, wp2_ref,
                bp_ref, h_ref, dinv_ref):
    x2 = jnp.dot(af_ref[...], wae_ref[...],
                 preferred_element_type=jnp.float32) + bae_ref[...]
    h_ref[...] = (jnp.dot(ax_ref[...], wp1_ref[...],
                          preferred_element_type=jnp.float32)
                  + jnp.dot(x2, wp2_ref[...],
                            preferred_element_type=jnp.float32)
                  + bp_ref[...])
    deg = degp_ref[0:1, :] + degp_ref[1:2, :] + 1.0
    dinv_ref[...] = lax.rsqrt(deg)


def _enc_call(ax_p, af_p, degp, W_ae, b_ae, Wp1, Wp2, b_proj):
    full = lambda shape: pl.BlockSpec(shape, lambda i: (0,) * len(shape))
    return pl.pallas_call(
        _enc_kernel,
        grid=(NP // _BLK,),
        in_specs=[
            pl.BlockSpec((_BLK, EMB), lambda i: (i, 0)),
            pl.BlockSpec((_BLK, 16), lambda i: (i, 0)),
            pl.BlockSpec((2, _BLK), lambda i: (0, i)),
            full((16, EMB)), full((1, EMB)), full((EMB, EMB)),
            full((EMB, EMB)), full((1, EMB)),
        ],
        out_specs=[
            pl.BlockSpec((_BLK, EMB), lambda i: (i, 0)),
            pl.BlockSpec((1, _BLK), lambda i: (0, i)),
        ],
        out_shape=[
            jax.ShapeDtypeStruct((NP, EMB), jnp.float32),
            jax.ShapeDtypeStruct((1, NP), jnp.float32),
        ],
    )(ax_p, af_p, degp, W_ae, b_ae, Wp1, Wp2, b_proj)


def _xs_kernel(h_ref, dinvb_ref, wl_ref, bl_ref, x_ref, xs_ref):
    x = jnp.dot(h_ref[...], wl_ref[...],
                preferred_element_type=jnp.float32) + bl_ref[...]
    x_ref[...] = x
    xs_ref[...] = x * dinvb_ref[...]


def _xs_call(h, dinvb, wl, bl):
    full = lambda shape: pl.BlockSpec(shape, lambda i: (0,) * len(shape))
    return pl.pallas_call(
        _xs_kernel,
        grid=(NP // _BLK,),
        in_specs=[
            pl.BlockSpec((_BLK, EMB), lambda i: (i, 0)),
            pl.BlockSpec((_BLK, EMB), lambda i: (i, 0)),
            full((EMB, EMB)), full((1, EMB)),
        ],
        out_specs=[pl.BlockSpec((_BLK, EMB), lambda i: (i, 0)),
                   pl.BlockSpec((_BLK, EMB), lambda i: (i, 0))],
        out_shape=[jax.ShapeDtypeStruct((NP, EMB), jnp.float32),
                   jax.ShapeDtypeStruct((NP, EMB), jnp.float32)],
    )(h, dinvb, wl, bl)


def _ee_kernel(eaA_ref, eaB_ref, wbe_ref, bbe_ref, g_ref, b_ref, m_ref,
               v_ref, ee_ref):
    scale = g_ref[...] * lax.rsqrt(v_ref[...] + BN_EPS_EDGE)
    w = wbe_ref[...] * scale
    cst = (bbe_ref[...] - m_ref[...]) * scale + b_ref[...]
    waug = jnp.concatenate([w, cst, jnp.zeros((AUG - 17, EMB), jnp.float32)],
                           axis=0)
    eeA = jnp.dot(eaA_ref[...], waug, preferred_element_type=jnp.float32)
    eeB = jnp.dot(eaB_ref[...], waug, preferred_element_type=jnp.float32)
    a16 = lax.bitcast_convert_type(eeA.astype(jnp.bfloat16), jnp.int16)
    b16 = lax.bitcast_convert_type(eeB.astype(jnp.bfloat16), jnp.int16)
    word = ((b16.astype(jnp.int32) << 16)
            | (a16.astype(jnp.int32) & 0xFFFF))
    ee_ref[...] = lax.bitcast_convert_type(word, jnp.float32)


def _ee_call(ea_aug, wbe, bbe, g, b, m, v):
    full = lambda shape: pl.BlockSpec(shape, lambda i: (0,) * len(shape))
    nblk = E // 2 // _EBLK
    return pl.pallas_call(
        _ee_kernel,
        grid=(nblk,),
        in_specs=[
            pl.BlockSpec((_EBLK, AUG), lambda i: (i, 0)),
            pl.BlockSpec((_EBLK, AUG), lambda i: (i + nblk, 0)),
            full((16, EMB)), full((1, EMB)), full((1, EMB)),
            full((1, EMB)), full((1, EMB)), full((1, EMB)),
        ],
        out_specs=pl.BlockSpec((_EBLK, EMB), lambda i: (i, 0)),
        out_shape=jax.ShapeDtypeStruct((E // 2, EMB), jnp.float32),
    )(ea_aug, ea_aug, wbe, bbe, g, b, m, v)


def _upd_kernel(relu_out,
                aggp_ref, x_ref, dinvb_ref, d2_ref, wih_ref, whh_ref, bih_ref,
                bhh_ref, root_ref, bng_ref, bnb_ref, bnm_ref, bnv_ref, h_ref):
    x = x_ref[...]
    aggr = (aggp_ref[0] + aggp_ref[1]) * dinvb_ref[...]
    gi = jnp.dot(aggr, wih_ref[...],
                 preferred_element_type=jnp.float32) + bih_ref[...]
    gh = jnp.dot(x, whh_ref[...],
                 preferred_element_type=jnp.float32) + bhh_ref[...]
    r = jax.nn.sigmoid(gi[:, :EMB] + gh[:, :EMB])
    z = jax.nn.sigmoid(gi[:, EMB:2 * EMB] + gh[:, EMB:2 * EMB])
    nn_ = jnp.tanh(gi[:, 2 * EMB:] + r * gh[:, 2 * EMB:])
    upd = (1.0 - z) * nn_ + z * x
    conv = upd + jnp.maximum(x + root_ref[...], 0.0) * d2_ref[...]
    hb = ((conv - bnm_ref[...]) * lax.rsqrt(bnv_ref[...] + BN_EPS_OUT)
          * bng_ref[...] + bnb_ref[...])
    if relu_out:
        hb = jnp.maximum(hb, 0.0)
    h_ref[...] = hb


def _upd_call(aggp, x, dinvb, d2b, wih, whh, bih, bhh, root, bng, bnb,
              bnm, bnv, relu_out):
    full = lambda shape: pl.BlockSpec(shape, lambda i: (0,) * len(shape))
    return pl.pallas_call(
        functools.partial(_upd_kernel, relu_out),
        grid=(NP // _BLK,),
        in_specs=[
            pl.BlockSpec((2, _BLK, EMB), lambda i: (0, i, 0)),
            pl.BlockSpec((_BLK, EMB), lambda i: (i, 0)),
            pl.BlockSpec((_BLK, EMB), lambda i: (i, 0)),
            pl.BlockSpec((_BLK, EMB), lambda i: (i, 0)),
            full((EMB, 3 * EMB)), full((EMB, 3 * EMB)),
            full((1, 3 * EMB)), full((1, 3 * EMB)),
            full((1, EMB)), full((1, EMB)), full((1, EMB)),
            full((1, EMB)), full((1, EMB)),
        ],
        out_specs=pl.BlockSpec((_BLK, EMB), lambda i: (i, 0)),
        out_shape=jax.ShapeDtypeStruct((NP, EMB), jnp.float32),
    )(aggp, x, dinvb, d2b, wih, whh, bih, bhh, root, bng, bnb, bnm, bnv)


def kernel(atom_x, atom_feature, edge_index, edge_attr,
           W_ae, b_ae, W_proj, b_proj,
           W_lin, b_lin, root_emb, W_be, b_be,
           bn_be_g, bn_be_b, bn_be_m, bn_be_v,
           W_ih, W_hh, b_ih, b_hh,
           bn_g, bn_b, bn_m, bn_v):
    pad_n = NP - N

    idx_all = jnp.concatenate(
        [edge_index[0].reshape(NCH, 1, CH), edge_index[1].reshape(NCH, 1, CH)],
        axis=1)

    def _pair(v):
        a = v[:E // 2].reshape(NCH, CH // 2, 1)
        b = v[E // 2:].reshape(NCH, CH // 2, 1)
        return jnp.concatenate([a, b], axis=2).reshape(NCH, 1, CH)
    idx_pair = jnp.concatenate(
        [_pair(edge_index[0]), _pair(edge_index[1])], axis=1)
    ax_p = jnp.pad(atom_x, ((0, pad_n), (0, 0)))
    af_p = jnp.pad(atom_feature, ((0, pad_n), (0, 0)))

    r2 = lambda a: a.reshape(1, -1)
    Wp1 = W_proj[:EMB]
    Wp2 = W_proj[EMB:]

    degp = _deg_call(idx_all).reshape(NC, NP)
    h, dinv2d = _enc_call(ax_p, af_p, degp, W_ae, r2(b_ae), Wp1, Wp2,
                          r2(b_proj))
    dinv = dinv2d.reshape(NP)
    dinvb = jnp.broadcast_to(dinv[:, None], (NP, EMB))
    d2b = jnp.broadcast_to((dinv * dinv)[:, None], (NP, EMB))

    ea_aug = _prep_call(idx_all, edge_attr, dinv)
    ee0 = _ee_call(ea_aug, W_be[0], r2(b_be[0]), r2(bn_be_g[0]),
                   r2(bn_be_b[0]), r2(bn_be_m[0]), r2(bn_be_v[0]))
    ee1 = _ee_call(ea_aug, W_be[1], r2(b_be[1]), r2(bn_be_g[1]),
                   r2(bn_be_b[1]), r2(bn_be_m[1]), r2(bn_be_v[1]))

    x0, xs0 = _xs_call(h, dinvb, W_lin[0], r2(b_lin[0]))
    aggp0 = _edge_call(idx_pair, xs0, ee0)
    h1 = _upd_call(aggp0, x0, dinvb, d2b, W_ih[0], W_hh[0], r2(b_ih[0]),
                   r2(b_hh[0]), r2(root_emb[0]), r2(bn_g[0]), r2(bn_b[0]),
                   r2(bn_m[0]), r2(bn_v[0]), relu_out=True)

    x1, xs1 = _xs_call(h1, dinvb, W_lin[1], r2(b_lin[1]))
    aggp1 = _edge_call(idx_pair, xs1, ee1)
    h2 = _upd_call(aggp1, x1, dinvb, d2b, W_ih[1], W_hh[1], r2(b_ih[1]),
                   r2(b_hh[1]), r2(root_emb[1]), r2(bn_g[1]), r2(bn_b[1]),
                   r2(bn_m[1]), r2(bn_v[1]), relu_out=False)

    return h2[:N]


# speedup vs baseline: 1.4021x; 1.0177x over previous
import functools

import jax
import jax.numpy as jnp
from jax import lax
from jax.experimental import pallas as pl
from jax.experimental.pallas import tpu as pltpu
from jax.experimental.pallas import tpu_sc as plsc

N = 10000
E = 320000
EMB = 128
NP = 10240
NC = 2
NS = 16
NW = NC * NS
CH = 80
G = 125
NCH = NW * G
RPT = NP // NS
AUG = 32
BN_EPS_EDGE = 1e-6
BN_EPS_OUT = 1e-5


def _deg_body(idx_ref, out_ref, ones_v, stage_v, idxA, idxB,
              semiA, semiB, semaA, semaB, deg_sh):
    c = lax.axis_index("c")
    s = lax.axis_index("s")
    wid = s * NC + c

    for i in range(CH // 16):
        ones_v[pl.ds(i * 16, 16)] = jnp.ones((16,), jnp.float32)

    def zero_stage(i, carry):
        stage_v[pl.ds(i * 16, 16)] = jnp.zeros((16,), jnp.float32)
        return carry
    lax.fori_loop(0, RPT // 16, zero_stage, 0)
    pltpu.sync_copy(stage_v, deg_sh.at[pl.ds(s * RPT, RPT)])
    plsc.subcore_barrier()

    def issue(g, idxv, semi, sema):
        cid = wid * G + g

        @pl.when(g >= 2)
        def _():
            pltpu.make_async_copy(ones_v, deg_sh.at[idxv.at[0]], sema).wait()
        pltpu.async_copy(idx_ref.at[cid], idxv, semi)

    def process(g, idxv, semi, sema):
        cid = wid * G + g
        pltpu.make_async_copy(idx_ref.at[cid], idxv, semi).wait()
        pltpu.async_copy(ones_v, deg_sh.at[idxv.at[0]], sema, add=True)

    issue(0, idxA, semiA, semaA)
    issue(1, idxB, semiB, semaB)

    def body(gg, carry):
        g0 = 2 * gg
        process(g0, idxA, semiA, semaA)

        @pl.when(g0 + 2 < G)
        def _():
            issue(g0 + 2, idxA, semiA, semaA)

        process(g0 + 1, idxB, semiB, semaB)

        @pl.when(g0 + 3 < G)
        def _():
            issue(g0 + 3, idxB, semiB, semaB)
        return carry
    lax.fori_loop(0, (G - 1) // 2, body, 0)
    process(G - 1, idxA, semiA, semaA)
    pltpu.make_async_copy(ones_v, deg_sh.at[idxA.at[0]], semaA).wait()
    pltpu.make_async_copy(ones_v, deg_sh.at[idxB.at[0]], semaB).wait()
    plsc.subcore_barrier()

    pltpu.sync_copy(deg_sh.at[pl.ds(s * RPT, RPT)], stage_v)
    pltpu.sync_copy(stage_v, out_ref.at[pl.ds(c * NP + s * RPT, RPT)])


def _deg_call(idx_all):
    mesh = plsc.VectorSubcoreMesh(core_axis_name="c", subcore_axis_name="s")
    f = pl.kernel(
        _deg_body,
        out_type=jax.ShapeDtypeStruct((NC * NP,), jnp.float32),
        mesh=mesh,
        compiler_params=pltpu.CompilerParams(needs_layout_passes=False),
        scratch_types=[
            pltpu.VMEM((CH,), jnp.float32),
            pltpu.VMEM((RPT,), jnp.float32),
            pltpu.VMEM((2, CH), jnp.int32),
            pltpu.VMEM((2, CH), jnp.int32),
            pltpu.SemaphoreType.DMA,
            pltpu.SemaphoreType.DMA,
            pltpu.SemaphoreType.DMA,
            pltpu.SemaphoreType.DMA,
            pltpu.VMEM_SHARED((NP,), jnp.float32),
        ],
    )
    return f(idx_all)


def _prep_body(idx_ref, ea_ref, dinv_ref, out_ref,
               dinv_v, idxA, idxB, eaA, eaB, augA, augB,
               semiA, semiB, semoA, semoB):
    c = lax.axis_index("c")
    s = lax.axis_index("s")
    wid = s * NC + c

    pltpu.sync_copy(dinv_ref, dinv_v)
    lane0 = jnp.where(lax.iota(jnp.int32, 16) == 0, 1.0, 0.0)

    def issue(g, idxv, eav, semi):
        cid = wid * G + g
        pltpu.sync_copy(idx_ref.at[cid], idxv)
        pltpu.async_copy(ea_ref.at[pl.ds(cid * CH, CH)], eav, semi)

    def process(g, idxv, eav, augv, semi, semo):
        cid = wid * G + g
        pltpu.make_async_copy(ea_ref.at[pl.ds(0, CH)], eav, semi).wait()

        @pl.when(g >= 2)
        def _():
            pltpu.make_async_copy(
                augv, out_ref.at[pl.ds(cid * CH, CH)], semo).wait()
        @plsc.parallel_loop(0, CH // 16, unroll=2)
        def scalegrp(k):
            ir = idxv[0, pl.ds(k * 16, 16)]
            dr = plsc.load_gather(dinv_v, [ir])
            for t in range(16):
                sc = dr[t]
                e = k * 16 + t
                augv[e, pl.ds(0, 16)] = eav[e, pl.ds(0, 16)] * sc
                augv[e, pl.ds(16, 16)] = lane0 * sc
        pltpu.async_copy(augv, out_ref.at[pl.ds(cid * CH, CH)], semo)

    issue(0, idxA, eaA, semiA)
    issue(1, idxB, eaB, semiB)

    def body(gg, carry):
        g0 = 2 * gg
        process(g0, idxA, eaA, augA, semiA, semoA)

        @pl.when(g0 + 2 < G)
        def _():
            issue(g0 + 2, idxA, eaA, semiA)

        process(g0 + 1, idxB, eaB, augB, semiB, semoB)

        @pl.when(g0 + 3 < G)
        def _():
            issue(g0 + 3, idxB, eaB, semiB)
        return carry
    lax.fori_loop(0, (G - 1) // 2, body, 0)
    process(G - 1, idxA, eaA, augA, semiA, semoA)
    pltpu.make_async_copy(augA, out_ref.at[pl.ds(0, CH)], semoA).wait()
    pltpu.make_async_copy(augB, out_ref.at[pl.ds(0, CH)], semoB).wait()


def _prep_call(idx_all, edge_attr, dinv):
    mesh = plsc.VectorSubcoreMesh(core_axis_name="c", subcore_axis_name="s")
    f = pl.kernel(
        _prep_body,
        out_type=jax.ShapeDtypeStruct((E, AUG), jnp.float32),
        mesh=mesh,
        compiler_params=pltpu.CompilerParams(needs_layout_passes=False),
        scratch_types=[
            pltpu.VMEM((NP,), jnp.float32),
            pltpu.VMEM((2, CH), jnp.int32),
            pltpu.VMEM((2, CH), jnp.int32),
            pltpu.VMEM((CH, 16), jnp.float32),
            pltpu.VMEM((CH, 16), jnp.float32),
            pltpu.VMEM((CH, AUG), jnp.float32),
            pltpu.VMEM((CH, AUG), jnp.float32),
            pltpu.SemaphoreType.DMA,
            pltpu.SemaphoreType.DMA,
            pltpu.SemaphoreType.DMA,
            pltpu.SemaphoreType.DMA,
        ],
    )
    return f(idx_all, edge_attr, dinv)


def _edge_body(idx_ref, x_ref, ee_ref, out_ref,
               idxA, idxB, xrA, xrB, eeA, eeB, msg_v,
               semxA, semxB, semeA, semeB, aggr_sh):
    c = lax.axis_index("c")
    s = lax.axis_index("s")
    wid = s * NC + c

    def zrow(i, carry):
        for j in range(EMB // 16):
            msg_v[i, pl.ds(j * 16, 16)] = jnp.zeros((16,), jnp.float32)
        return carry
    lax.fori_loop(0, CH, zrow, 0)

    def zslab(k, carry):
        pltpu.sync_copy(msg_v, aggr_sh.at[pl.ds(s * RPT + k * CH, CH)])
        return carry
    lax.fori_loop(0, RPT // CH, zslab, 0)
    plsc.subcore_barrier()

    def issue(g, idxv, xrv, eev, semx, seme):
        cid = wid * G + g
        pltpu.sync_copy(idx_ref.at[cid], idxv)
        pltpu.async_copy(x_ref.at[idxv.at[0]], xrv, semx)
        pltpu.async_copy(ee_ref.at[pl.ds(cid * (CH // 2), CH // 2)], eev, seme)

    def process(idxv, xrv, eev, semx, seme):
        pltpu.make_async_copy(x_ref.at[idxv.at[0]], xrv, semx).wait()
        pltpu.make_async_copy(ee_ref.at[pl.ds(0, CH // 2)], eev, seme).wait()

        @plsc.parallel_loop(0, CH // 2, unroll=4)
        def pairrow(r):
            for j in range(EMB // 16):
                pw = eev[r, pl.ds(j * 16, 16)]
                pb = plsc.bitcast(pw, jnp.bfloat16)
                ea_, eb_ = plsc.unpack(pb, format=plsc.PackFormat.INTERLEAVED)
                xa = xrv[2 * r, pl.ds(j * 16, 16)]
                xb = xrv[2 * r + 1, pl.ds(j * 16, 16)]
                msg_v[2 * r, pl.ds(j * 16, 16)] = jnp.maximum(xa + ea_, 0.0)
                msg_v[2 * r + 1, pl.ds(j * 16, 16)] = jnp.maximum(xb + eb_, 0.0)
        pltpu.sync_copy(msg_v, aggr_sh.at[idxv.at[1]], add=True)

    issue(0, idxA, xrA, eeA, semxA, semeA)
    issue(1, idxB, xrB, eeB, semxB, semeB)

    def body(gg, carry):
        g0 = 2 * gg
        process(idxA, xrA, eeA, semxA, semeA)

        @pl.when(g0 + 2 < G)
        def _():
            issue(g0 + 2, idxA, xrA, eeA, semxA, semeA)

        process(idxB, xrB, eeB, semxB, semeB)

        @pl.when(g0 + 3 < G)
        def _():
            issue(g0 + 3, idxB, xrB, eeB, semxB, semeB)
        return carry
    lax.fori_loop(0, (G - 1) // 2, body, 0)
    process(idxA, xrA, eeA, semxA, semeA)

    plsc.subcore_barrier()

    def wout(k, carry):
        off = s * RPT + k * CH
        pltpu.sync_copy(aggr_sh.at[pl.ds(off, CH)], msg_v)
        pltpu.sync_copy(msg_v, out_ref.at[c, pl.ds(off, CH)])
        return carry
    lax.fori_loop(0, RPT // CH, wout, 0)


def _edge_call(idx_all, xs, ee):
    mesh = plsc.VectorSubcoreMesh(core_axis_name="c", subcore_axis_name="s")
    f = pl.kernel(
        _edge_body,
        out_type=jax.ShapeDtypeStruct((NC, NP, EMB), jnp.float32),
        mesh=mesh,
        compiler_params=pltpu.CompilerParams(needs_layout_passes=False),
        scratch_types=[
            pltpu.VMEM((2, CH), jnp.int32),
            pltpu.VMEM((2, CH), jnp.int32),
            pltpu.VMEM((CH, EMB), jnp.float32),
            pltpu.VMEM((CH, EMB), jnp.float32),
            pltpu.VMEM((CH // 2, EMB), jnp.float32),
            pltpu.VMEM((CH // 2, EMB), jnp.float32),
            pltpu.VMEM((CH, EMB), jnp.float32),
            pltpu.SemaphoreType.DMA,
            pltpu.SemaphoreType.DMA,
            pltpu.SemaphoreType.DMA,
            pltpu.SemaphoreType.DMA,
            pltpu.VMEM_SHARED((NP, EMB), jnp.float32),
        ],
    )
    return f(idx_all, xs, ee)


_BLK = 1024
_EBLK = 1280


def _enc_kernel(ax_ref, af_ref, wae_ref, bae_ref, wp1_ref, wp2_ref,
                bp_ref, h_ref):
    x2 = jnp.dot(af_ref[...], wae_ref[...],
                 preferred_element_type=jnp.float32) + bae_ref[...]
    h_ref[...] = (jnp.dot(ax_ref[...], wp1_ref[...],
                          preferred_element_type=jnp.float32)
                  + jnp.dot(x2, wp2_ref[...],
                            preferred_element_type=jnp.float32)
                  + bp_ref[...])


def _enc_call(ax_p, af_p, W_ae, b_ae, Wp1, Wp2, b_proj):
    full = lambda shape: pl.BlockSpec(shape, lambda i: (0,) * len(shape))
    return pl.pallas_call(
        _enc_kernel,
        grid=(NP // _BLK,),
        in_specs=[
            pl.BlockSpec((_BLK, EMB), lambda i: (i, 0)),
            pl.BlockSpec((_BLK, 16), lambda i: (i, 0)),
            full((16, EMB)), full((1, EMB)), full((EMB, EMB)),
            full((EMB, EMB)), full((1, EMB)),
        ],
        out_specs=pl.BlockSpec((_BLK, EMB), lambda i: (i, 0)),
        out_shape=jax.ShapeDtypeStruct((NP, EMB), jnp.float32),
    )(ax_p, af_p, W_ae, b_ae, Wp1, Wp2, b_proj)


def _dinv_kernel(degp_ref, dinv_ref):
    deg = degp_ref[0:1, :] + degp_ref[1:2, :] + 1.0
    dinv_ref[...] = lax.rsqrt(deg)


def _dinv_call(degp):
    return pl.pallas_call(
        _dinv_kernel,
        grid=(NP // _BLK,),
        in_specs=[pl.BlockSpec((2, _BLK), lambda i: (0, i))],
        out_specs=pl.BlockSpec((1, _BLK), lambda i: (0, i)),
        out_shape=jax.ShapeDtypeStruct((1, NP), jnp.float32),
    )(degp)


def _xs_kernel(h_ref, dinvb_ref, wl_ref, bl_ref, x_ref, xs_ref):
    x = jnp.dot(h_ref[...], wl_ref[...],
                preferred_element_type=jnp.float32) + bl_ref[...]
    x_ref[...] = x
    xs_ref[...] = x * dinvb_ref[...]


def _xs_call(h, dinvb, wl, bl):
    full = lambda shape: pl.BlockSpec(shape, lambda i: (0,) * len(shape))
    return pl.pallas_call(
        _xs_kernel,
        grid=(NP // _BLK,),
        in_specs=[
            pl.BlockSpec((_BLK, EMB), lambda i: (i, 0)),
            pl.BlockSpec((_BLK, EMB), lambda i: (i, 0)),
            full((EMB, EMB)), full((1, EMB)),
        ],
        out_specs=[pl.BlockSpec((_BLK, EMB), lambda i: (i, 0)),
                   pl.BlockSpec((_BLK, EMB), lambda i: (i, 0))],
        out_shape=[jax.ShapeDtypeStruct((NP, EMB), jnp.float32),
                   jax.ShapeDtypeStruct((NP, EMB), jnp.float32)],
    )(h, dinvb, wl, bl)


def _ee_kernel(eaA_ref, eaB_ref, wbe_ref, bbe_ref, g_ref, b_ref, m_ref,
               v_ref, ee_ref):
    scale = g_ref[...] * lax.rsqrt(v_ref[...] + BN_EPS_EDGE)
    w = wbe_ref[...] * scale
    cst = (bbe_ref[...] - m_ref[...]) * scale + b_ref[...]
    waug = jnp.concatenate([w, cst, jnp.zeros((AUG - 17, EMB), jnp.float32)],
                           axis=0)
    eeA = jnp.dot(eaA_ref[...], waug, preferred_element_type=jnp.float32)
    eeB = jnp.dot(eaB_ref[...], waug, preferred_element_type=jnp.float32)
    a16 = lax.bitcast_convert_type(eeA.astype(jnp.bfloat16), jnp.int16)
    b16 = lax.bitcast_convert_type(eeB.astype(jnp.bfloat16), jnp.int16)
    word = ((b16.astype(jnp.int32) << 16)
            | (a16.astype(jnp.int32) & 0xFFFF))
    ee_ref[...] = lax.bitcast_convert_type(word, jnp.float32)


def _ee_call(ea_aug, wbe, bbe, g, b, m, v):
    full = lambda shape: pl.BlockSpec(shape, lambda i: (0,) * len(shape))
    nblk = E // 2 // _EBLK
    return pl.pallas_call(
        _ee_kernel,
        grid=(nblk,),
        in_specs=[
            pl.BlockSpec((_EBLK, AUG), lambda i: (i, 0)),
            pl.BlockSpec((_EBLK, AUG), lambda i: (i + nblk, 0)),
            full((16, EMB)), full((1, EMB)), full((1, EMB)),
            full((1, EMB)), full((1, EMB)), full((1, EMB)),
        ],
        out_specs=pl.BlockSpec((_EBLK, EMB), lambda i: (i, 0)),
        out_shape=jax.ShapeDtypeStruct((E // 2, EMB), jnp.float32),
    )(ea_aug, ea_aug, wbe, bbe, g, b, m, v)


def _upd_kernel(relu_out,
                aggp_ref, x_ref, dinvb_ref, d2_ref, wih_ref, whh_ref, bih_ref,
                bhh_ref, root_ref, bng_ref, bnb_ref, bnm_ref, bnv_ref, h_ref):
    x = x_ref[...]
    aggr = (aggp_ref[0] + aggp_ref[1]) * dinvb_ref[...]
    gi = jnp.dot(aggr, wih_ref[...],
                 preferred_element_type=jnp.float32) + bih_ref[...]
    gh = jnp.dot(x, whh_ref[...],
                 preferred_element_type=jnp.float32) + bhh_ref[...]
    r = jax.nn.sigmoid(gi[:, :EMB] + gh[:, :EMB])
    z = jax.nn.sigmoid(gi[:, EMB:2 * EMB] + gh[:, EMB:2 * EMB])
    nn_ = jnp.tanh(gi[:, 2 * EMB:] + r * gh[:, 2 * EMB:])
    upd = (1.0 - z) * nn_ + z * x
    conv = upd + jnp.maximum(x + root_ref[...], 0.0) * d2_ref[...]
    hb = ((conv - bnm_ref[...]) * lax.rsqrt(bnv_ref[...] + BN_EPS_OUT)
          * bng_ref[...] + bnb_ref[...])
    if relu_out:
        hb = jnp.maximum(hb, 0.0)
    h_ref[...] = hb


def _upd_call(aggp, x, dinvb, d2b, wih, whh, bih, bhh, root, bng, bnb,
              bnm, bnv, relu_out):
    full = lambda shape: pl.BlockSpec(shape, lambda i: (0,) * len(shape))
    return pl.pallas_call(
        functools.partial(_upd_kernel, relu_out),
        grid=(NP // _BLK,),
        in_specs=[
            pl.BlockSpec((2, _BLK, EMB), lambda i: (0, i, 0)),
            pl.BlockSpec((_BLK, EMB), lambda i: (i, 0)),
            pl.BlockSpec((_BLK, EMB), lambda i: (i, 0)),
            pl.BlockSpec((_BLK, EMB), lambda i: (i, 0)),
            full((EMB, 3 * EMB)), full((EMB, 3 * EMB)),
            full((1, 3 * EMB)), full((1, 3 * EMB)),
            full((1, EMB)), full((1, EMB)), full((1, EMB)),
            full((1, EMB)), full((1, EMB)),
        ],
        out_specs=pl.BlockSpec((_BLK, EMB), lambda i: (i, 0)),
        out_shape=jax.ShapeDtypeStruct((NP, EMB), jnp.float32),
    )(aggp, x, dinvb, d2b, wih, whh, bih, bhh, root, bng, bnb, bnm, bnv)


def kernel(atom_x, atom_feature, edge_index, edge_attr,
           W_ae, b_ae, W_proj, b_proj,
           W_lin, b_lin, root_emb, W_be, b_be,
           bn_be_g, bn_be_b, bn_be_m, bn_be_v,
           W_ih, W_hh, b_ih, b_hh,
           bn_g, bn_b, bn_m, bn_v):
    pad_n = NP - N

    idx_all = jnp.concatenate(
        [edge_index[0].reshape(NCH, 1, CH), edge_index[1].reshape(NCH, 1, CH)],
        axis=1)

    def _pair(v):
        a = v[:E // 2].reshape(NCH, CH // 2, 1)
        b = v[E // 2:].reshape(NCH, CH // 2, 1)
        return jnp.concatenate([a, b], axis=2).reshape(NCH, 1, CH)
    idx_pair = jnp.concatenate(
        [_pair(edge_index[0]), _pair(edge_index[1])], axis=1)
    ax_p = jnp.pad(atom_x, ((0, pad_n), (0, 0)))
    af_p = jnp.pad(atom_feature, ((0, pad_n), (0, 0)))

    r2 = lambda a: a.reshape(1, -1)
    Wp1 = W_proj[:EMB]
    Wp2 = W_proj[EMB:]

    degp = _deg_call(idx_all).reshape(NC, NP)
    h = _enc_call(ax_p, af_p, W_ae, r2(b_ae), Wp1, Wp2, r2(b_proj))
    dinv = _dinv_call(degp).reshape(NP)
    dinvb = jnp.broadcast_to(dinv[:, None], (NP, EMB))
    d2b = jnp.broadcast_to((dinv * dinv)[:, None], (NP, EMB))

    ea_aug = _prep_call(idx_all, edge_attr, dinv)
    ee0 = _ee_call(ea_aug, W_be[0], r2(b_be[0]), r2(bn_be_g[0]),
                   r2(bn_be_b[0]), r2(bn_be_m[0]), r2(bn_be_v[0]))
    ee1 = _ee_call(ea_aug, W_be[1], r2(b_be[1]), r2(bn_be_g[1]),
                   r2(bn_be_b[1]), r2(bn_be_m[1]), r2(bn_be_v[1]))

    x0, xs0 = _xs_call(h, dinvb, W_lin[0], r2(b_lin[0]))
    aggp0 = _edge_call(idx_pair, xs0, ee0)
    h1 = _upd_call(aggp0, x0, dinvb, d2b, W_ih[0], W_hh[0], r2(b_ih[0]),
                   r2(b_hh[0]), r2(root_emb[0]), r2(bn_g[0]), r2(bn_b[0]),
                   r2(bn_m[0]), r2(bn_v[0]), relu_out=True)

    x1, xs1 = _xs_call(h1, dinvb, W_lin[1], r2(b_lin[1]))
    aggp1 = _edge_call(idx_pair, xs1, ee1)
    h2 = _upd_call(aggp1, x1, dinvb, d2b, W_ih[1], W_hh[1], r2(b_ih[1]),
                   r2(b_hh[1]), r2(root_emb[1]), r2(bn_g[1]), r2(bn_b[1]),
                   r2(bn_m[1]), r2(bn_v[1]), relu_out=False)

    return h2[:N]


# speedup vs baseline: 1.4933x; 1.0651x over previous
import functools

import jax
import jax.numpy as jnp
from jax import lax
from jax.experimental import pallas as pl
from jax.experimental.pallas import tpu as pltpu
from jax.experimental.pallas import tpu_sc as plsc

N = 10000
E = 320000
EMB = 128
NP = 10240
NC = 2
NS = 16
NW = NC * NS
CH = 80
G = 125
NCH = NW * G
RPT = NP // NS
AUG = 32
BN_EPS_EDGE = 1e-6
BN_EPS_OUT = 1e-5


def _deg_body(idx_ref, out_ref, ones_v, stage_v, idxA, idxB,
              semiA, semiB, semaA, semaB, deg_sh):
    c = lax.axis_index("c")
    s = lax.axis_index("s")
    wid = s * NC + c

    for i in range(CH // 16):
        ones_v[pl.ds(i * 16, 16)] = jnp.ones((16,), jnp.float32)

    def zero_stage(i, carry):
        stage_v[pl.ds(i * 16, 16)] = jnp.zeros((16,), jnp.float32)
        return carry
    lax.fori_loop(0, RPT // 16, zero_stage, 0)
    pltpu.sync_copy(stage_v, deg_sh.at[pl.ds(s * RPT, RPT)])
    plsc.subcore_barrier()

    def issue(g, idxv, semi, sema):
        cid = wid * G + g

        @pl.when(g >= 2)
        def _():
            pltpu.make_async_copy(ones_v, deg_sh.at[idxv.at[0]], sema).wait()
        pltpu.async_copy(idx_ref.at[cid], idxv, semi)

    def process(g, idxv, semi, sema):
        cid = wid * G + g
        pltpu.make_async_copy(idx_ref.at[cid], idxv, semi).wait()
        pltpu.async_copy(ones_v, deg_sh.at[idxv.at[0]], sema, add=True)

    issue(0, idxA, semiA, semaA)
    issue(1, idxB, semiB, semaB)

    def body(gg, carry):
        g0 = 2 * gg
        process(g0, idxA, semiA, semaA)

        @pl.when(g0 + 2 < G)
        def _():
            issue(g0 + 2, idxA, semiA, semaA)

        process(g0 + 1, idxB, semiB, semaB)

        @pl.when(g0 + 3 < G)
        def _():
            issue(g0 + 3, idxB, semiB, semaB)
        return carry
    lax.fori_loop(0, (G - 1) // 2, body, 0)
    process(G - 1, idxA, semiA, semaA)
    pltpu.make_async_copy(ones_v, deg_sh.at[idxA.at[0]], semaA).wait()
    pltpu.make_async_copy(ones_v, deg_sh.at[idxB.at[0]], semaB).wait()
    plsc.subcore_barrier()

    pltpu.sync_copy(deg_sh.at[pl.ds(s * RPT, RPT)], stage_v)
    pltpu.sync_copy(stage_v, out_ref.at[pl.ds(c * NP + s * RPT, RPT)])


def _deg_call(idx_all):
    mesh = plsc.VectorSubcoreMesh(core_axis_name="c", subcore_axis_name="s")
    f = pl.kernel(
        _deg_body,
        out_type=jax.ShapeDtypeStruct((NC * NP,), jnp.float32),
        mesh=mesh,
        compiler_params=pltpu.CompilerParams(needs_layout_passes=False),
        scratch_types=[
            pltpu.VMEM((CH,), jnp.float32),
            pltpu.VMEM((RPT,), jnp.float32),
            pltpu.VMEM((2, CH), jnp.int32),
            pltpu.VMEM((2, CH), jnp.int32),
            pltpu.SemaphoreType.DMA,
            pltpu.SemaphoreType.DMA,
            pltpu.SemaphoreType.DMA,
            pltpu.SemaphoreType.DMA,
            pltpu.VMEM_SHARED((NP,), jnp.float32),
        ],
    )
    return f(idx_all)


def _prep_body(idx_ref, ea_ref, dinv_ref, out_ref,
               dinv_v, idxA, idxB, eaA, eaB, augA, augB,
               semiA, semiB, semoA, semoB):
    c = lax.axis_index("c")
    s = lax.axis_index("s")
    wid = s * NC + c

    pltpu.sync_copy(dinv_ref, dinv_v)
    lane0 = jnp.where(lax.iota(jnp.int32, 16) == 0, 1.0, 0.0)

    def issue(g, idxv, eav, semi):
        cid = wid * G + g
        pltpu.sync_copy(idx_ref.at[cid], idxv)
        pltpu.async_copy(ea_ref.at[pl.ds(cid * CH, CH)], eav, semi)

    def process(g, idxv, eav, augv, semi, semo):
        cid = wid * G + g
        pltpu.make_async_copy(ea_ref.at[pl.ds(0, CH)], eav, semi).wait()

        @pl.when(g >= 2)
        def _():
            pltpu.make_async_copy(
                augv, out_ref.at[pl.ds(cid * CH, CH)], semo).wait()
        @plsc.parallel_loop(0, CH // 16, unroll=2)
        def scalegrp(k):
            ir = idxv[0, pl.ds(k * 16, 16)]
            dr = plsc.load_gather(dinv_v, [ir])
            for t in range(16):
                sc = dr[t]
                e = k * 16 + t
                augv[e, pl.ds(0, 16)] = eav[e, pl.ds(0, 16)] * sc
                augv[e, pl.ds(16, 16)] = lane0 * sc
        pltpu.async_copy(augv, out_ref.at[pl.ds(cid * CH, CH)], semo)

    issue(0, idxA, eaA, semiA)
    issue(1, idxB, eaB, semiB)

    def body(gg, carry):
        g0 = 2 * gg
        process(g0, idxA, eaA, augA, semiA, semoA)

        @pl.when(g0 + 2 < G)
        def _():
            issue(g0 + 2, idxA, eaA, semiA)

        process(g0 + 1, idxB, eaB, augB, semiB, semoB)

        @pl.when(g0 + 3 < G)
        def _():
            issue(g0 + 3, idxB, eaB, semiB)
        return carry
    lax.fori_loop(0, (G - 1) // 2, body, 0)
    process(G - 1, idxA, eaA, augA, semiA, semoA)
    pltpu.make_async_copy(augA, out_ref.at[pl.ds(0, CH)], semoA).wait()
    pltpu.make_async_copy(augB, out_ref.at[pl.ds(0, CH)], semoB).wait()


def _prep_call(idx_all, edge_attr, dinv):
    mesh = plsc.VectorSubcoreMesh(core_axis_name="c", subcore_axis_name="s")
    f = pl.kernel(
        _prep_body,
        out_type=jax.ShapeDtypeStruct((E, AUG), jnp.float32),
        mesh=mesh,
        compiler_params=pltpu.CompilerParams(needs_layout_passes=False),
        scratch_types=[
            pltpu.VMEM((NP,), jnp.float32),
            pltpu.VMEM((2, CH), jnp.int32),
            pltpu.VMEM((2, CH), jnp.int32),
            pltpu.VMEM((CH, 16), jnp.float32),
            pltpu.VMEM((CH, 16), jnp.float32),
            pltpu.VMEM((CH, AUG), jnp.float32),
            pltpu.VMEM((CH, AUG), jnp.float32),
            pltpu.SemaphoreType.DMA,
            pltpu.SemaphoreType.DMA,
            pltpu.SemaphoreType.DMA,
            pltpu.SemaphoreType.DMA,
        ],
    )
    return f(idx_all, edge_attr, dinv)


def _edge_body(idx_ref, x_ref, ee_ref, out_ref,
               idx0, idx1, idx2, xr0, xr1, xr2, ee0, ee1, ee2,
               semx0, semx1, semx2, seme0, seme1, seme2,
               sems0, sems1, sems2, aggr_sh):
    c = lax.axis_index("c")
    s = lax.axis_index("s")
    wid = s * NC + c

    def zrow(i, carry):
        for j in range(EMB // 16):
            xr0[i, pl.ds(j * 16, 16)] = jnp.zeros((16,), jnp.float32)
        return carry
    lax.fori_loop(0, CH, zrow, 0)

    def zslab(k, carry):
        pltpu.sync_copy(xr0, aggr_sh.at[pl.ds(s * RPT + k * CH, CH)])
        return carry
    lax.fori_loop(0, RPT // CH, zslab, 0)
    plsc.subcore_barrier()

    def issue(g, idxv, xrv, eev, semx, seme):
        cid = wid * G + g
        pltpu.sync_copy(idx_ref.at[cid], idxv)
        pltpu.async_copy(x_ref.at[idxv.at[0]], xrv, semx)
        pltpu.async_copy(ee_ref.at[pl.ds(cid * (CH // 2), CH // 2)], eev, seme)

    def process(idxv, xrv, eev, semx, seme, sems):
        pltpu.make_async_copy(x_ref.at[idxv.at[0]], xrv, semx).wait()
        pltpu.make_async_copy(ee_ref.at[pl.ds(0, CH // 2)], eev, seme).wait()

        @plsc.parallel_loop(0, CH // 2, unroll=4)
        def pairrow(r):
            for j in range(EMB // 16):
                pw = eev[r, pl.ds(j * 16, 16)]
                pb = plsc.bitcast(pw, jnp.bfloat16)
                ea_, eb_ = plsc.unpack(pb, format=plsc.PackFormat.INTERLEAVED)
                xa = xrv[2 * r, pl.ds(j * 16, 16)]
                xb = xrv[2 * r + 1, pl.ds(j * 16, 16)]
                xrv[2 * r, pl.ds(j * 16, 16)] = jnp.maximum(xa + ea_, 0.0)
                xrv[2 * r + 1, pl.ds(j * 16, 16)] = jnp.maximum(xb + eb_, 0.0)
        pltpu.async_copy(xrv, aggr_sh.at[idxv.at[1]], sems, add=True)

    def drain(idxv, xrv, sems):
        pltpu.make_async_copy(xrv, aggr_sh.at[idxv.at[1]], sems).wait()

    issue(0, idx0, xr0, ee0, semx0, seme0)
    issue(1, idx1, xr1, ee1, semx1, seme1)

    def body(gg, carry):
        g0 = 3 * gg
        process(idx0, xr0, ee0, semx0, seme0, sems0)

        @pl.when(g0 > 0)
        def _():
            drain(idx2, xr2, sems2)
        issue(g0 + 2, idx2, xr2, ee2, semx2, seme2)

        process(idx1, xr1, ee1, semx1, seme1, sems1)
        drain(idx0, xr0, sems0)
        issue(g0 + 3, idx0, xr0, ee0, semx0, seme0)

        process(idx2, xr2, ee2, semx2, seme2, sems2)
        drain(idx1, xr1, sems1)
        issue(g0 + 4, idx1, xr1, ee1, semx1, seme1)
        return carry
    lax.fori_loop(0, (G - 2) // 3, body, 0)
    process(idx0, xr0, ee0, semx0, seme0, sems0)
    process(idx1, xr1, ee1, semx1, seme1, sems1)
    drain(idx2, xr2, sems2)
    drain(idx0, xr0, sems0)
    drain(idx1, xr1, sems1)

    plsc.subcore_barrier()

    def wout(k, carry):
        off = s * RPT + k * CH
        pltpu.sync_copy(aggr_sh.at[pl.ds(off, CH)], xr0)
        pltpu.sync_copy(xr0, out_ref.at[c, pl.ds(off, CH)])
        return carry
    lax.fori_loop(0, RPT // CH, wout, 0)


def _edge_call(idx_all, xs, ee):
    mesh = plsc.VectorSubcoreMesh(core_axis_name="c", subcore_axis_name="s")
    f = pl.kernel(
        _edge_body,
        out_type=jax.ShapeDtypeStruct((NC, NP, EMB), jnp.float32),
        mesh=mesh,
        compiler_params=pltpu.CompilerParams(needs_layout_passes=False),
        scratch_types=[
            pltpu.VMEM((2, CH), jnp.int32),
            pltpu.VMEM((2, CH), jnp.int32),
            pltpu.VMEM((2, CH), jnp.int32),
            pltpu.VMEM((CH, EMB), jnp.float32),
            pltpu.VMEM((CH, EMB), jnp.float32),
            pltpu.VMEM((CH, EMB), jnp.float32),
            pltpu.VMEM((CH // 2, EMB), jnp.float32),
            pltpu.VMEM((CH // 2, EMB), jnp.float32),
            pltpu.VMEM((CH // 2, EMB), jnp.float32),
            pltpu.SemaphoreType.DMA,
            pltpu.SemaphoreType.DMA,
            pltpu.SemaphoreType.DMA,
            pltpu.SemaphoreType.DMA,
            pltpu.SemaphoreType.DMA,
            pltpu.SemaphoreType.DMA,
            pltpu.SemaphoreType.DMA,
            pltpu.SemaphoreType.DMA,
            pltpu.SemaphoreType.DMA,
            pltpu.VMEM_SHARED((NP, EMB), jnp.float32),
        ],
    )
    return f(idx_all, xs, ee)


_BLK = 1024
_EBLK = 1280


def _enc_kernel(ax_ref, af_ref, wae_ref, bae_ref, wp1_ref, wp2_ref,
                bp_ref, h_ref):
    x2 = jnp.dot(af_ref[...], wae_ref[...],
                 preferred_element_type=jnp.float32) + bae_ref[...]
    h_ref[...] = (jnp.dot(ax_ref[...], wp1_ref[...],
                          preferred_element_type=jnp.float32)
                  + jnp.dot(x2, wp2_ref[...],
                            preferred_element_type=jnp.float32)
                  + bp_ref[...])


def _enc_call(ax_p, af_p, W_ae, b_ae, Wp1, Wp2, b_proj):
    full = lambda shape: pl.BlockSpec(shape, lambda i: (0,) * len(shape))
    return pl.pallas_call(
        _enc_kernel,
        grid=(NP // _BLK,),
        in_specs=[
            pl.BlockSpec((_BLK, EMB), lambda i: (i, 0)),
            pl.BlockSpec((_BLK, 16), lambda i: (i, 0)),
            full((16, EMB)), full((1, EMB)), full((EMB, EMB)),
            full((EMB, EMB)), full((1, EMB)),
        ],
        out_specs=pl.BlockSpec((_BLK, EMB), lambda i: (i, 0)),
        out_shape=jax.ShapeDtypeStruct((NP, EMB), jnp.float32),
    )(ax_p, af_p, W_ae, b_ae, Wp1, Wp2, b_proj)


def _dinv_kernel(degp_ref, dinv_ref):
    deg = degp_ref[0:1, :] + degp_ref[1:2, :] + 1.0
    dinv_ref[...] = lax.rsqrt(deg)


def _dinv_call(degp):
    return pl.pallas_call(
        _dinv_kernel,
        grid=(NP // _BLK,),
        in_specs=[pl.BlockSpec((2, _BLK), lambda i: (0, i))],
        out_specs=pl.BlockSpec((1, _BLK), lambda i: (0, i)),
        out_shape=jax.ShapeDtypeStruct((1, NP), jnp.float32),
    )(degp)


def _xs_kernel(h_ref, dinvb_ref, wl_ref, bl_ref, x_ref, xs_ref):
    x = jnp.dot(h_ref[...], wl_ref[...],
                preferred_element_type=jnp.float32) + bl_ref[...]
    x_ref[...] = x
    xs_ref[...] = x * dinvb_ref[...]


def _xs_call(h, dinvb, wl, bl):
    full = lambda shape: pl.BlockSpec(shape, lambda i: (0,) * len(shape))
    return pl.pallas_call(
        _xs_kernel,
        grid=(NP // _BLK,),
        in_specs=[
            pl.BlockSpec((_BLK, EMB), lambda i: (i, 0)),
            pl.BlockSpec((_BLK, EMB), lambda i: (i, 0)),
            full((EMB, EMB)), full((1, EMB)),
        ],
        out_specs=[pl.BlockSpec((_BLK, EMB), lambda i: (i, 0)),
                   pl.BlockSpec((_BLK, EMB), lambda i: (i, 0))],
        out_shape=[jax.ShapeDtypeStruct((NP, EMB), jnp.float32),
                   jax.ShapeDtypeStruct((NP, EMB), jnp.float32)],
    )(h, dinvb, wl, bl)


def _ee_kernel(eaA_ref, eaB_ref, wbe_ref, bbe_ref, g_ref, b_ref, m_ref,
               v_ref, ee_ref):
    scale = g_ref[...] * lax.rsqrt(v_ref[...] + BN_EPS_EDGE)
    w = wbe_ref[...] * scale
    cst = (bbe_ref[...] - m_ref[...]) * scale + b_ref[...]
    waug = jnp.concatenate([w, cst, jnp.zeros((AUG - 17, EMB), jnp.float32)],
                           axis=0)
    eeA = jnp.dot(eaA_ref[...], waug, preferred_element_type=jnp.float32)
    eeB = jnp.dot(eaB_ref[...], waug, preferred_element_type=jnp.float32)
    a16 = lax.bitcast_convert_type(eeA.astype(jnp.bfloat16), jnp.int16)
    b16 = lax.bitcast_convert_type(eeB.astype(jnp.bfloat16), jnp.int16)
    word = ((b16.astype(jnp.int32) << 16)
            | (a16.astype(jnp.int32) & 0xFFFF))
    ee_ref[...] = lax.bitcast_convert_type(word, jnp.float32)


def _ee_call(ea_aug, wbe, bbe, g, b, m, v):
    full = lambda shape: pl.BlockSpec(shape, lambda i: (0,) * len(shape))
    nblk = E // 2 // _EBLK
    return pl.pallas_call(
        _ee_kernel,
        grid=(nblk,),
        in_specs=[
            pl.BlockSpec((_EBLK, AUG), lambda i: (i, 0)),
            pl.BlockSpec((_EBLK, AUG), lambda i: (i + nblk, 0)),
            full((16, EMB)), full((1, EMB)), full((1, EMB)),
            full((1, EMB)), full((1, EMB)), full((1, EMB)),
        ],
        out_specs=pl.BlockSpec((_EBLK, EMB), lambda i: (i, 0)),
        out_shape=jax.ShapeDtypeStruct((E // 2, EMB), jnp.float32),
    )(ea_aug, ea_aug, wbe, bbe, g, b, m, v)


def _upd_kernel(relu_out,
                aggp_ref, x_ref, dinvb_ref, d2_ref, wih_ref, whh_ref, bih_ref,
                bhh_ref, root_ref, bng_ref, bnb_ref, bnm_ref, bnv_ref, h_ref):
    x = x_ref[...]
    aggr = (aggp_ref[0] + aggp_ref[1]) * dinvb_ref[...]
    gi = jnp.dot(aggr, wih_ref[...],
                 preferred_element_type=jnp.float32) + bih_ref[...]
    gh = jnp.dot(x, whh_ref[...],
                 preferred_element_type=jnp.float32) + bhh_ref[...]
    r = jax.nn.sigmoid(gi[:, :EMB] + gh[:, :EMB])
    z = jax.nn.sigmoid(gi[:, EMB:2 * EMB] + gh[:, EMB:2 * EMB])
    nn_ = jnp.tanh(gi[:, 2 * EMB:] + r * gh[:, 2 * EMB:])
    upd = (1.0 - z) * nn_ + z * x
    conv = upd + jnp.maximum(x + root_ref[...], 0.0) * d2_ref[...]
    hb = ((conv - bnm_ref[...]) * lax.rsqrt(bnv_ref[...] + BN_EPS_OUT)
          * bng_ref[...] + bnb_ref[...])
    if relu_out:
        hb = jnp.maximum(hb, 0.0)
    h_ref[...] = hb


def _upd_call(aggp, x, dinvb, d2b, wih, whh, bih, bhh, root, bng, bnb,
              bnm, bnv, relu_out):
    full = lambda shape: pl.BlockSpec(shape, lambda i: (0,) * len(shape))
    return pl.pallas_call(
        functools.partial(_upd_kernel, relu_out),
        grid=(NP // _BLK,),
        in_specs=[
            pl.BlockSpec((2, _BLK, EMB), lambda i: (0, i, 0)),
            pl.BlockSpec((_BLK, EMB), lambda i: (i, 0)),
            pl.BlockSpec((_BLK, EMB), lambda i: (i, 0)),
            pl.BlockSpec((_BLK, EMB), lambda i: (i, 0)),
            full((EMB, 3 * EMB)), full((EMB, 3 * EMB)),
            full((1, 3 * EMB)), full((1, 3 * EMB)),
            full((1, EMB)), full((1, EMB)), full((1, EMB)),
            full((1, EMB)), full((1, EMB)),
        ],
        out_specs=pl.BlockSpec((_BLK, EMB), lambda i: (i, 0)),
        out_shape=jax.ShapeDtypeStruct((NP, EMB), jnp.float32),
    )(aggp, x, dinvb, d2b, wih, whh, bih, bhh, root, bng, bnb, bnm, bnv)


def kernel(atom_x, atom_feature, edge_index, edge_attr,
           W_ae, b_ae, W_proj, b_proj,
           W_lin, b_lin, root_emb, W_be, b_be,
           bn_be_g, bn_be_b, bn_be_m, bn_be_v,
           W_ih, W_hh, b_ih, b_hh,
           bn_g, bn_b, bn_m, bn_v):
    pad_n = NP - N

    idx_all = jnp.concatenate(
        [edge_index[0].reshape(NCH, 1, CH), edge_index[1].reshape(NCH, 1, CH)],
        axis=1)

    def _pair(v):
        a = v[:E // 2].reshape(NCH, CH // 2, 1)
        b = v[E // 2:].reshape(NCH, CH // 2, 1)
        return jnp.concatenate([a, b], axis=2).reshape(NCH, 1, CH)
    idx_pair = jnp.concatenate(
        [_pair(edge_index[0]), _pair(edge_index[1])], axis=1)
    ax_p = jnp.pad(atom_x, ((0, pad_n), (0, 0)))
    af_p = jnp.pad(atom_feature, ((0, pad_n), (0, 0)))

    r2 = lambda a: a.reshape(1, -1)
    Wp1 = W_proj[:EMB]
    Wp2 = W_proj[EMB:]

    degp = _deg_call(idx_all).reshape(NC, NP)
    h = _enc_call(ax_p, af_p, W_ae, r2(b_ae), Wp1, Wp2, r2(b_proj))
    dinv = _dinv_call(degp).reshape(NP)
    dinvb = jnp.broadcast_to(dinv[:, None], (NP, EMB))
    d2b = jnp.broadcast_to((dinv * dinv)[:, None], (NP, EMB))

    ea_aug = _prep_call(idx_all, edge_attr, dinv)
    ee0 = _ee_call(ea_aug, W_be[0], r2(b_be[0]), r2(bn_be_g[0]),
                   r2(bn_be_b[0]), r2(bn_be_m[0]), r2(bn_be_v[0]))
    ee1 = _ee_call(ea_aug, W_be[1], r2(b_be[1]), r2(bn_be_g[1]),
                   r2(bn_be_b[1]), r2(bn_be_m[1]), r2(bn_be_v[1]))

    x0, xs0 = _xs_call(h, dinvb, W_lin[0], r2(b_lin[0]))
    aggp0 = _edge_call(idx_pair, xs0, ee0)
    h1 = _upd_call(aggp0, x0, dinvb, d2b, W_ih[0], W_hh[0], r2(b_ih[0]),
                   r2(b_hh[0]), r2(root_emb[0]), r2(bn_g[0]), r2(bn_b[0]),
                   r2(bn_m[0]), r2(bn_v[0]), relu_out=True)

    x1, xs1 = _xs_call(h1, dinvb, W_lin[1], r2(b_lin[1]))
    aggp1 = _edge_call(idx_pair, xs1, ee1)
    h2 = _upd_call(aggp1, x1, dinvb, d2b, W_ih[1], W_hh[1], r2(b_ih[1]),
                   r2(b_hh[1]), r2(root_emb[1]), r2(bn_g[1]), r2(bn_b[1]),
                   r2(bn_m[1]), r2(bn_v[1]), relu_out=False)

    return h2[:N]
